# baseline (speedup 1.0000x reference)
import functools

import jax
import jax.numpy as jnp
from jax import lax
from jax.experimental import pallas as pl
from jax.experimental.pallas import tpu as pltpu
from jax.experimental.pallas import tpu_sc as plsc

N = 10000
E = 320000
D = 128
H = 128
OUT = 128
L = 2
TAU = 0.5

NP = 10112
NC = 2
NS = 16
NW = NC * NS
EW = E // NW
CK = 80
EWP = 10240
NCH = EWP // CK
CKA = 80
NCHA = EWP // CKA
CKU = 80
NCHU = EW // CKU
EP = NW * EWP
ZR = NP // NS
PAD_ROW = NP - 8

BN = 1000
BE = 2048


def _sc_mesh():
    return plsc.VectorSubcoreMesh(
        core_axis_name="c", subcore_axis_name="s", num_cores=NC,
        num_subcores=NS)


def _spmm_call(y, rowi, coli, zero):
    d = y.shape[1]

    def body(y_hbm, rowi_hbm, coli_hbm, zero_hbm, out_hbm,
             acc_sh, rbuf0, rbuf1, cbuf0, cbuf1, buf0, buf1, sem0, sem1):
        c = lax.axis_index("c")
        s = lax.axis_index("s")
        w = c * NS + s
        pltpu.sync_copy(zero_hbm, acc_sh.at[pl.ds(s * ZR, ZR)])
        plsc.subcore_barrier()

        j_first = w * 0
        pltpu.sync_copy(coli_hbm.at[w, j_first], cbuf0)
        pltpu.async_copy(y_hbm.at[cbuf0], buf0, sem0)

        def chunk2(g, carry):
            j0 = 2 * g
            j1 = j0 + 1
            j2 = j0 + 2
            pltpu.sync_copy(coli_hbm.at[w, j1], cbuf1)
            pltpu.async_copy(y_hbm.at[cbuf1], buf1, sem1)
            pltpu.sync_copy(rowi_hbm.at[w, j0], rbuf0)
            pltpu.make_async_copy(y_hbm.at[pl.ds(0, CK)], buf0, sem0).wait()
            pltpu.sync_copy(buf0, acc_sh.at[rbuf0], add=True)

            @pl.when(j2 < NCH)
            def _():
                pltpu.sync_copy(coli_hbm.at[w, j2], cbuf0)
                pltpu.async_copy(y_hbm.at[cbuf0], buf0, sem0)

            pltpu.sync_copy(rowi_hbm.at[w, j1], rbuf1)
            pltpu.make_async_copy(y_hbm.at[pl.ds(0, CK)], buf1, sem1).wait()
            pltpu.sync_copy(buf1, acc_sh.at[rbuf1], add=True)
            return carry

        lax.fori_loop(0, NCH // 2, chunk2, 0)
        plsc.subcore_barrier()
        pltpu.sync_copy(acc_sh.at[pl.ds(s * ZR, ZR)],
                        out_hbm.at[c, pl.ds(s * ZR, ZR)])

    f = pl.kernel(
        body,
        out_type=jax.ShapeDtypeStruct((NC, NP, d), jnp.float32),
        mesh=_sc_mesh(),
        scratch_types=[
            pltpu.VMEM_SHARED((NP, d), jnp.float32),
            pltpu.VMEM((CK,), jnp.int32),
            pltpu.VMEM((CK,), jnp.int32),
            pltpu.VMEM((CK,), jnp.int32),
            pltpu.VMEM((CK,), jnp.int32),
            pltpu.VMEM((CK, d), jnp.float32),
            pltpu.VMEM((CK, d), jnp.float32),
            pltpu.SemaphoreType.DMA,
            pltpu.SemaphoreType.DMA,
        ],
    )
    return f(y, rowi, coli, zero)


def _att_gather_call(A, B, XA, rowi, coli):
    def body(a_hbm, b_hbm, xa_hbm, rowi_hbm, coli_hbm,
             ga_hbm, gb_hbm, g_hbm,
             rb0, rb1, cb0, cb1, bufa0, bufb0, bufg0, bufa1, bufb1, bufg1,
             sa0, sb0, sg0, sa1, sb1, sg1):
        c = lax.axis_index("c")
        s = lax.axis_index("s")
        w = c * NS + s

        def issue0(j):
            pltpu.sync_copy(rowi_hbm.at[w, j], rb0)
            pltpu.sync_copy(coli_hbm.at[w, j], cb0)
            pltpu.async_copy(a_hbm.at[rb0], bufa0, sa0)
            pltpu.async_copy(b_hbm.at[cb0], bufb0, sb0)
            pltpu.async_copy(xa_hbm.at[cb0], bufg0, sg0)

        def issue1(j):
            pltpu.sync_copy(rowi_hbm.at[w, j], rb1)
            pltpu.sync_copy(coli_hbm.at[w, j], cb1)
            pltpu.async_copy(a_hbm.at[rb1], bufa1, sa1)
            pltpu.async_copy(b_hbm.at[cb1], bufb1, sb1)
            pltpu.async_copy(xa_hbm.at[cb1], bufg1, sg1)

        j_first = w * 0
        issue0(j_first)

        def chunk2(g, carry):
            j0 = 2 * g
            j1 = j0 + 1
            j2 = j0 + 2
            issue1(j1)
            base0 = pl.multiple_of(w * EWP + j0 * CKA, CKA)
            pltpu.make_async_copy(a_hbm.at[pl.ds(0, CKA)], bufa0, sa0).wait()
            pltpu.sync_copy(bufa0, ga_hbm.at[pl.ds(base0, CKA)])
            pltpu.make_async_copy(b_hbm.at[pl.ds(0, CKA)], bufb0, sb0).wait()
            pltpu.sync_copy(bufb0, gb_hbm.at[pl.ds(base0, CKA)])
            pltpu.make_async_copy(xa_hbm.at[pl.ds(0, CKA)], bufg0, sg0).wait()
            pltpu.sync_copy(bufg0, g_hbm.at[pl.ds(base0, CKA)])

            @pl.when(j2 < NCHA)
            def _():
                issue0(j2)

            base1 = pl.multiple_of(w * EWP + j1 * CKA, CKA)
            pltpu.make_async_copy(a_hbm.at[pl.ds(0, CKA)], bufa1, sa1).wait()
            pltpu.sync_copy(bufa1, ga_hbm.at[pl.ds(base1, CKA)])
            pltpu.make_async_copy(b_hbm.at[pl.ds(0, CKA)], bufb1, sb1).wait()
            pltpu.sync_copy(bufb1, gb_hbm.at[pl.ds(base1, CKA)])
            pltpu.make_async_copy(xa_hbm.at[pl.ds(0, CKA)], bufg1, sg1).wait()
            pltpu.sync_copy(bufg1, g_hbm.at[pl.ds(base1, CKA)])
            return carry

        lax.fori_loop(0, NCHA // 2, chunk2, 0)

    f = pl.kernel(
        body,
        out_type=(jax.ShapeDtypeStruct((EP, D), jnp.float32),
                  jax.ShapeDtypeStruct((EP, D), jnp.float32),
                  jax.ShapeDtypeStruct((EP, D), jnp.float32)),
        mesh=_sc_mesh(),
        scratch_types=[
            pltpu.VMEM((CKA,), jnp.int32),
            pltpu.VMEM((CKA,), jnp.int32),
            pltpu.VMEM((CKA,), jnp.int32),
            pltpu.VMEM((CKA,), jnp.int32),
            pltpu.VMEM((CKA, D), jnp.float32),
            pltpu.VMEM((CKA, D), jnp.float32),
            pltpu.VMEM((CKA, D), jnp.float32),
            pltpu.VMEM((CKA, D), jnp.float32),
            pltpu.VMEM((CKA, D), jnp.float32),
            pltpu.VMEM((CKA, D), jnp.float32),
            pltpu.SemaphoreType.DMA,
            pltpu.SemaphoreType.DMA,
            pltpu.SemaphoreType.DMA,
            pltpu.SemaphoreType.DMA,
            pltpu.SemaphoreType.DMA,
            pltpu.SemaphoreType.DMA,
        ],
    )
    return f(A, B, XA, rowi, coli)


def _scatter_call(vals, rowi, zero):
    d = vals.shape[1]

    def body(val_hbm, rowi_hbm, zero_hbm, out_hbm,
             acc_sh, rbuf0, rbuf1, buf0, buf1, sem0, sem1):
        c = lax.axis_index("c")
        s = lax.axis_index("s")
        w = c * NS + s
        pltpu.sync_copy(zero_hbm, acc_sh.at[pl.ds(s * ZR, ZR)])
        plsc.subcore_barrier()

        base_w = pl.multiple_of(w * EWP, CK)
        pltpu.async_copy(val_hbm.at[pl.ds(base_w, CK)], buf0, sem0)

        def chunk2(g, carry):
            j0 = 2 * g
            j1 = j0 + 1
            j2 = j0 + 2
            base1 = pl.multiple_of(w * EWP + j1 * CK, CK)
            pltpu.async_copy(val_hbm.at[pl.ds(base1, CK)], buf1, sem1)
            pltpu.sync_copy(rowi_hbm.at[w, j0], rbuf0)
            pltpu.make_async_copy(val_hbm.at[pl.ds(0, CK)], buf0, sem0).wait()
            pltpu.sync_copy(buf0, acc_sh.at[rbuf0], add=True)

            @pl.when(j2 < NCH)
            def _():
                base2 = pl.multiple_of(w * EWP + j2 * CK, CK)
                pltpu.async_copy(val_hbm.at[pl.ds(base2, CK)], buf0, sem0)

            pltpu.sync_copy(rowi_hbm.at[w, j1], rbuf1)
            pltpu.make_async_copy(val_hbm.at[pl.ds(0, CK)], buf1, sem1).wait()
            pltpu.sync_copy(buf1, acc_sh.at[rbuf1], add=True)
            return carry

        lax.fori_loop(0, NCH // 2, chunk2, 0)
        plsc.subcore_barrier()
        pltpu.sync_copy(acc_sh.at[pl.ds(s * ZR, ZR)],
                        out_hbm.at[c, pl.ds(s * ZR, ZR)])

    f = pl.kernel(
        body,
        out_type=jax.ShapeDtypeStruct((NC, NP, d), jnp.float32),
        mesh=_sc_mesh(),
        scratch_types=[
            pltpu.VMEM_SHARED((NP, d), jnp.float32),
            pltpu.VMEM((CK,), jnp.int32),
            pltpu.VMEM((CK,), jnp.int32),
            pltpu.VMEM((CK, d), jnp.float32),
            pltpu.VMEM((CK, d), jnp.float32),
            pltpu.SemaphoreType.DMA,
            pltpu.SemaphoreType.DMA,
        ],
    )
    return f(vals, rowi, zero)


def _lin(x, W, b, act=None):
    n, din = x.shape
    dout = W.shape[0]
    bn = BN if n == N else BE

    def body(x_ref, w_ref, b_ref, o_ref):
        y = lax.dot_general(x_ref[...], w_ref[...], (((1,), (1,)), ((), ())),
                            preferred_element_type=jnp.float32)
        y = y + b_ref[...]
        if act == "relu":
            y = jnp.maximum(y, 0.0)
        elif act == "sigmoid":
            y = jax.nn.sigmoid(y)
        o_ref[...] = y

    return pl.pallas_call(
        body,
        grid=(n // bn,),
        in_specs=[
            pl.BlockSpec((bn, din), lambda i: (i, 0)),
            pl.BlockSpec((dout, din), lambda i: (0, 0)),
            pl.BlockSpec((1, dout), lambda i: (0, 0)),
        ],
        out_specs=pl.BlockSpec((bn, dout), lambda i: (i, 0)),
        out_shape=jax.ShapeDtypeStruct((n, dout), jnp.float32),
    )(x, W, b.reshape(1, dout))


def _combine(parts, fulls, rows, fn, n=N, d=D, n_out=1, bn=None):
    if bn is None:
        bn = BN if n == N else BE
    nf = len(fulls)
    nr = len(rows)

    def body(*refs):
        k = 0
        args = []
        if parts is not None:
            args.append(refs[0][...][0] + refs[1][...][0])
            k = 2
        for r in refs[k:k + nf + nr]:
            args.append(r[...])
        outs = refs[k + nf + nr:]
        res = fn(*args)
        if n_out == 1:
            res = (res,)
        for o, v in zip(outs, res):
            o[...] = v

    in_specs = []
    ops = []
    if parts is not None:
        in_specs.append(pl.BlockSpec((1, bn, d), lambda i: (0, i, 0)))
        in_specs.append(pl.BlockSpec((1, bn, d), lambda i: (1, i, 0)))
        ops += [parts, parts]
    for a in fulls:
        in_specs.append(pl.BlockSpec((bn, d), lambda i: (i, 0)))
        ops.append(a)
    for a in rows:
        in_specs.append(pl.BlockSpec((1, d), lambda i: (0, 0)))
        ops.append(a)
    out_shape = [jax.ShapeDtypeStruct((n, d), jnp.float32)] * n_out
    out_specs = [pl.BlockSpec((bn, d), lambda i: (i, 0))] * n_out
    res = pl.pallas_call(
        body, grid=(n // bn,), in_specs=in_specs, out_specs=out_specs,
        out_shape=out_shape)(*ops)
    return res[0] if n_out == 1 else res


def _stats_call(x, degb):
    def body(x_ref, d_ref, o_ref):
        i = pl.program_id(0)
        xb = x_ref[...]
        db = d_ref[...]
        blk = jnp.concatenate([
            jnp.sum(xb, axis=0, keepdims=True),
            jnp.sum(xb * xb, axis=0, keepdims=True),
            jnp.sum(db, axis=0, keepdims=True),
            jnp.sum(db * db, axis=0, keepdims=True),
            jnp.sum((db == 0.0).astype(jnp.float32), axis=0, keepdims=True),
            jnp.zeros((3, 128), jnp.float32),
        ], axis=0)

        @pl.when(i == 0)
        def _():
            o_ref[...] = blk

        @pl.when(i > 0)
        def _():
            o_ref[...] = o_ref[...] + blk

    return pl.pallas_call(
        body,
        grid=(N // BN,),
        in_specs=[pl.BlockSpec((BN, 128), lambda i: (i, 0)),
                  pl.BlockSpec((BN, 128), lambda i: (i, 0))],
        out_specs=pl.BlockSpec((8, 128), lambda i: (0, 0)),
        out_shape=jax.ShapeDtypeStruct((8, 128), jnp.float32),
    )(x, degb)


def _ctrl_call(h0p, W1p, b1, W2p, b2p):
    def body(h_ref, w1_ref, b1_ref, w2_ref, b2_ref, o_ref):
        r1 = lax.dot_general(h_ref[...], w1_ref[...], (((1,), (1,)), ((), ())),
                             preferred_element_type=jnp.float32) + b1_ref[...]
        r1 = jnp.maximum(r1, 0.0)
        o_ref[...] = lax.dot_general(
            r1, w2_ref[...], (((1,), (1,)), ((), ())),
            preferred_element_type=jnp.float32) + b2_ref[...]

    return pl.pallas_call(
        body,
        out_shape=jax.ShapeDtypeStruct((8, 128), jnp.float32),
    )(h0p, W1p, b1, W2p, b2p)


def _edge_score_call(GA, GB, W2, b2):
    def body(c_ref, a_ref, b_ref, w_ref, o_ref):
        r = jnp.maximum(a_ref[...] + b_ref[...], 0.0)
        o_ref[...] = lax.dot_general(
            r, w_ref[...], (((1,), (1,)), ((), ())),
            preferred_element_type=jnp.float32) + c_ref[0]

    return pl.pallas_call(
        body,
        grid=(EP // BE,),
        in_specs=[
            pl.BlockSpec(memory_space=pltpu.SMEM),
            pl.BlockSpec((BE, 128), lambda i: (i, 0)),
            pl.BlockSpec((BE, 128), lambda i: (i, 0)),
            pl.BlockSpec((8, 128), lambda i: (0, 0)),
        ],
        out_specs=pl.BlockSpec((BE, 8), lambda i: (i, 0)),
        out_shape=jax.ShapeDtypeStruct((EP, 8), jnp.float32),
    )(b2, GA, GB, W2)


def _redmax_call(a, mask):
    n, d = a.shape

    def body(a_ref, k_ref, o_ref):
        m = a_ref[...] * k_ref[...] - (1.0 - k_ref[...]) * 1e30
        o_ref[...] = jnp.max(m, axis=0, keepdims=True)

    return pl.pallas_call(
        body,
        out_shape=jax.ShapeDtypeStruct((1, d), jnp.float32))(a, mask)


def _redsumexp_call(a, mxr, mask):
    n, d = a.shape

    def body(a_ref, m_ref, k_ref, o_ref):
        o_ref[...] = jnp.sum(jnp.exp(a_ref[...] - m_ref[...]) * k_ref[...],
                             axis=0, keepdims=True)

    return pl.pallas_call(
        body,
        out_shape=jax.ShapeDtypeStruct((1, d), jnp.float32))(a, mxr, mask)


def _wmul_call(attn1, G):
    def body(a_ref, g_ref, o_ref):
        o_ref[...] = a_ref[...] * g_ref[...]

    return pl.pallas_call(
        body, grid=(EP // BE,),
        in_specs=[pl.BlockSpec((BE, 1), lambda i: (i, 0)),
                  pl.BlockSpec((BE, 128), lambda i: (i, 0))],
        out_specs=pl.BlockSpec((BE, 128), lambda i: (i, 0)),
        out_shape=jax.ShapeDtypeStruct((EP, 128), jnp.float32))(attn1, G)


def _row(v):
    return jnp.full((1, 128), 1.0, jnp.float32) * v


def kernel(edge_index, x, prev_emb, ctrl_W1, ctrl_b1, ctrl_W2, ctrl_b2,
           mode_bias, att_W1, att_b1, att_W2, att_b2, heat_kernels, time_W,
           time_b, scale_weights, sp_W, sp_b, tm_W, tm_b, at_W, at_b, df_W,
           df_b, hr_W, hr_b, out_W1, out_b1, out_W2, out_b2):
    row = edge_index[0]
    col = edge_index[1]
    rowp = jnp.full((NW, EWP), PAD_ROW, jnp.int32).at[:, :EW].set(
        row.reshape(NW, EW))
    colp = jnp.zeros((NW, EWP), jnp.int32).at[:, :EW].set(
        col.reshape(NW, EW))
    rowi = rowp.reshape(NW, NCH, CK)
    coli = colp.reshape(NW, NCH, CK)
    rowia = rowp.reshape(NW, NCHA, CKA)
    colia = colp.reshape(NW, NCHA, CKA)
    rowiu = row.reshape(NW, NCHU, CKU)
    coliu = col.reshape(NW, NCHU, CKU)
    ke = jnp.arange(EP, dtype=jnp.int32)
    emask = (ke % EWP < EW).astype(jnp.float32)
    mask2d = emask.reshape(EP // 128, 128)
    zero = jnp.zeros((ZR, D), jnp.float32)

    Sdeg = _spmm_call(jnp.ones((N, D), jnp.float32), rowi, coli, zero)
    degb, degm, dinv = _combine(
        Sdeg, [], [],
        lambda p: (p, jnp.maximum(p, 1.0),
                   jnp.maximum(lax.rsqrt(p), 1e-8)),
        n_out=3)

    acc = _stats_call(x, degb)
    s_x = jnp.sum(acc[0])
    s_x2 = jnp.sum(acc[1])
    s_d = acc[2, 0]
    s_d2 = acc[3, 0]
    s_z = acc[4, 0]
    cnt = float(N * D)
    mean_x = s_x / cnt
    std_x = jnp.sqrt(jnp.maximum((s_x2 - cnt * mean_x * mean_x) / (cnt - 1.0),
                                 0.0))
    mean_d = s_d / N
    std_d = jnp.sqrt(jnp.maximum((s_d2 - N * mean_d * mean_d) / (N - 1.0),
                                 0.0))
    stats = jnp.stack([
        jnp.float32(N / 1000.0), jnp.float32(E / max(N, 1)), std_d, s_z / N,
        mean_x, std_x, jnp.float32(1.0), jnp.float32(E / (N * N)),
    ])
    quality = jnp.mean(prev_emb, axis=0)
    h0 = jnp.concatenate([stats, quality])
    h0p = jnp.zeros((8, 256), jnp.float32).at[0, :8 + H].set(h0)
    W1p = jnp.zeros((128, 256), jnp.float32).at[:, :8 + H].set(ctrl_W1)
    W2p = jnp.zeros((128, 128), jnp.float32).at[:5].set(ctrl_W2)
    b2p = jnp.zeros((1, 128), jnp.float32).at[0, :5].set(ctrl_b2)
    logits = _ctrl_call(h0p, W1p, ctrl_b1.reshape(1, 128), W2p, b2p)[0, :5]
    logits = logits + mode_bias
    u = jax.random.uniform(jax.random.key(42), (5,), dtype=jnp.float32)
    g = -jnp.log(-jnp.log(u + 1e-20) + 1e-20)
    probs = jax.nn.softmax((logits + g) / TAU)

    xs = x
    for i in range(L):
        y = _lin(xs, sp_W[i], sp_b[i])
        S = _spmm_call(y, rowi, coli, zero)
        xs = _combine(S, [degm], [],
                      lambda p, dm: jnp.maximum(p / dm, 0.0))

    xt = x
    tW = time_W[:, :H]
    for i in range(L):
        xt1 = _lin(xt, tm_W[i], tm_b[i])
        gate = _lin(xt1, tW, time_b, act="sigmoid")
        S = _spmm_call(xt1, rowi, coli, zero)
        xt = _combine(S, [xt1, gate, degm], [],
                      lambda p, z, gt, dm: jnp.maximum(
                          gt * z + (1.0 - gt) * (p / dm), 0.0))

    xa = x
    W1a = jnp.zeros((128, H), jnp.float32).at[:64].set(att_W1[:, :H])
    W1b = jnp.zeros((128, H), jnp.float32).at[:64].set(att_W1[:, H:])
    b1p = jnp.zeros((128,), jnp.float32).at[:64].set(att_b1)
    W2p = jnp.zeros((8, 128), jnp.float32).at[:1, :64].set(att_W2)
    for i in range(L):
        xa = _lin(xa, at_W[i], at_b[i])
        A = _lin(xa, W1a, b1p)
        B = _lin(xa, W1b, jnp.zeros((128,), jnp.float32))
        GA, GB, G = _att_gather_call(A, B, xa, rowia, colia)
        sc = _edge_score_call(GA, GB, W2p, att_b2)
        sc2d = sc[:, 0].reshape(EP // 128, 128)
        mx = jnp.max(_redmax_call(sc2d, mask2d))
        ssum = jnp.sum(_redsumexp_call(sc2d, _row(mx), mask2d))
        attn2d = _combine(None, [sc2d, mask2d], [_row(mx), _row(1.0 / ssum)],
                          lambda a, k, m, r: jnp.exp(a - m) * k * r,
                          n=EP // 128, d=128, bn=EP // 128)
        WG = _wmul_call(attn2d.reshape(EP, 1), G)
        S = _scatter_call(WG, rowi, zero)
        xa = _combine(S, [], [], lambda p: jnp.maximum(p, 0.0))

    xd = x
    for i in range(L):
        xd = _lin(xd, df_W[i], df_b[i])
        acc_d = jnp.zeros((N, D), jnp.float32)
        y = _combine(None, [xd, dinv], [], lambda a, b: a * b)
        for k in range(5):
            t = heat_kernels[k]
            S = _spmm_call(y, rowi, coli, zero)
            xd, y, acc_d = _combine(
                S, [xd, acc_d, dinv], [_row(t)],
                lambda p, xo, ao, dv, tr: (
                    (1.0 - tr) * xo + tr * (dv * p),
                    dv * ((1.0 - tr) * xo + tr * (dv * p)),
                    ao + (1.0 - tr) * xo + tr * (dv * p)),
                n_out=3)
        xd = _combine(None, [acc_d], [],
                      lambda a: jnp.maximum(a * (1.0 / 5.0), 0.0))

    w3 = jax.nn.softmax(scale_weights)
    xh = x
    for i in range(L):
        xh = _lin(xh, hr_W[i], hr_b[i])
        S1 = _spmm_call(xh, rowi, coli, zero)
        h1 = _combine(S1, [degm], [], lambda p, dm: p / dm)
        S2 = _spmm_call(h1, rowi, coli, zero)
        h2 = _combine(S2, [degm], [], lambda p, dm: p / dm)
        S3 = _spmm_call(h2, rowi, coli, zero)
        xh = _combine(S3, [h1, h2, degm], [_row(w3[0]), _row(w3[1]),
                                           _row(w3[2])],
                      lambda p, a, b, dm, w0, w1, w2: jnp.maximum(
                          w0 * a + w1 * b + w2 * (p / dm), 0.0))

    xout = _combine(None, [xs, xt, xa, xd, xh],
                    [_row(probs[0]), _row(probs[1]), _row(probs[2]),
                     _row(probs[3]), _row(probs[4])],
                    lambda a, b, c, d_, e, p0, p1, p2, p3, p4:
                    p0 * a + p1 * b + p2 * c + p3 * d_ + p4 * e)
    o1 = _lin(xout, out_W1, out_b1, act="relu")
    return _lin(o1, out_W2, out_b2)


# speedup vs baseline: 2.1987x; 2.1987x over previous
import functools

import jax
import jax.numpy as jnp
from jax import lax
from jax.experimental import pallas as pl
from jax.experimental.pallas import tpu as pltpu
from jax.experimental.pallas import tpu_sc as plsc

N = 10000
E = 320000
D = 128
H = 128
OUT = 128
L = 2
TAU = 0.5

NP = 10112
NC = 2
NS = 16
NW = NC * NS
EW = E // NW
CK = 128
EWP = 10240
NCH = EWP // CK
CKA = 128
NCHA = EWP // CKA
CKU = 80
NCHU = EW // CKU
EP = NW * EWP
ZR = NP // NS
PAD_ROW = NP - 8

BN = 1000
BE = 2048


def _sc_mesh():
    return plsc.VectorSubcoreMesh(
        core_axis_name="c", subcore_axis_name="s", num_cores=NC,
        num_subcores=NS)


def _spmm_call(y, rowi, coli, zero):
    d = y.shape[1]

    def body(y_hbm, rowi_hbm, coli_hbm, zero_hbm, out_hbm,
             acc_sh, rbuf0, rbuf1, cbuf0, cbuf1, buf0, buf1, sem0, sem1):
        c = lax.axis_index("c")
        s = lax.axis_index("s")
        w = c * NS + s
        pltpu.sync_copy(zero_hbm, acc_sh.at[pl.ds(s * ZR, ZR)])
        plsc.subcore_barrier()

        j_first = w * 0
        pltpu.sync_copy(coli_hbm.at[w, j_first], cbuf0)
        pltpu.async_copy(y_hbm.at[cbuf0], buf0, sem0)
        pltpu.sync_copy(rowi_hbm.at[w, j_first], rbuf0)
        pltpu.make_async_copy(y_hbm.at[pl.ds(0, CKU)], buf0, sem0).wait()
        pltpu.sync_copy(buf0, acc_sh.at[rbuf0], add=True)
        pltpu.sync_copy(coli_hbm.at[w, j_first + 1], cbuf0)
        pltpu.async_copy(y_hbm.at[cbuf0], buf0, sem0)

        def chunk2(g, carry):
            j0 = 2 * g + 1
            j1 = j0 + 1
            j2 = j0 + 2
            pltpu.sync_copy(coli_hbm.at[w, j1], cbuf1)
            pltpu.async_copy(y_hbm.at[cbuf1], buf1, sem1)
            pltpu.sync_copy(rowi_hbm.at[w, j0], rbuf0)
            pltpu.make_async_copy(y_hbm.at[pl.ds(0, CKU)], buf0, sem0).wait()
            pltpu.sync_copy(buf0, acc_sh.at[rbuf0], add=True)

            @pl.when(j2 < NCHU)
            def _():
                pltpu.sync_copy(coli_hbm.at[w, j2], cbuf0)
                pltpu.async_copy(y_hbm.at[cbuf0], buf0, sem0)

            pltpu.sync_copy(rowi_hbm.at[w, j1], rbuf1)
            pltpu.make_async_copy(y_hbm.at[pl.ds(0, CKU)], buf1, sem1).wait()
            pltpu.sync_copy(buf1, acc_sh.at[rbuf1], add=True)
            return carry

        lax.fori_loop(0, (NCHU - 1) // 2, chunk2, 0)
        plsc.subcore_barrier()
        pltpu.sync_copy(acc_sh.at[pl.ds(s * ZR, ZR)],
                        out_hbm.at[c, pl.ds(s * ZR, ZR)])

    f = pl.kernel(
        body,
        out_type=jax.ShapeDtypeStruct((NC, NP, d), jnp.float32),
        mesh=_sc_mesh(),
        scratch_types=[
            pltpu.VMEM_SHARED((NP, d), jnp.float32),
            pltpu.VMEM((CKU,), jnp.int32),
            pltpu.VMEM((CKU,), jnp.int32),
            pltpu.VMEM((CKU,), jnp.int32),
            pltpu.VMEM((CKU,), jnp.int32),
            pltpu.VMEM((CKU, d), jnp.float32),
            pltpu.VMEM((CKU, d), jnp.float32),
            pltpu.SemaphoreType.DMA,
            pltpu.SemaphoreType.DMA,
        ],
    )
    return f(y, rowi, coli, zero)


def _att_gather_call(A, B, XA, rowi, coli):
    def body(a_hbm, b_hbm, xa_hbm, rowi_hbm, coli_hbm,
             ga_hbm, gb_hbm, g_hbm,
             rb0, rb1, cb0, cb1, bufa0, bufb0, bufg0, bufa1, bufb1, bufg1,
             sa0, sb0, sg0, sa1, sb1, sg1):
        c = lax.axis_index("c")
        s = lax.axis_index("s")
        w = c * NS + s

        def issue0(j):
            pltpu.sync_copy(rowi_hbm.at[w, j], rb0)
            pltpu.sync_copy(coli_hbm.at[w, j], cb0)
            pltpu.async_copy(a_hbm.at[rb0], bufa0, sa0)
            pltpu.async_copy(b_hbm.at[cb0], bufb0, sb0)
            pltpu.async_copy(xa_hbm.at[cb0], bufg0, sg0)

        def issue1(j):
            pltpu.sync_copy(rowi_hbm.at[w, j], rb1)
            pltpu.sync_copy(coli_hbm.at[w, j], cb1)
            pltpu.async_copy(a_hbm.at[rb1], bufa1, sa1)
            pltpu.async_copy(b_hbm.at[cb1], bufb1, sb1)
            pltpu.async_copy(xa_hbm.at[cb1], bufg1, sg1)

        j_first = w * 0
        issue0(j_first)

        def chunk2(g, carry):
            j0 = 2 * g
            j1 = j0 + 1
            j2 = j0 + 2
            issue1(j1)
            base0 = pl.multiple_of(w * EWP + j0 * CKA, CKA)
            pltpu.make_async_copy(a_hbm.at[pl.ds(0, CKA)], bufa0, sa0).wait()
            pltpu.sync_copy(bufa0, ga_hbm.at[pl.ds(base0, CKA)])
            pltpu.make_async_copy(b_hbm.at[pl.ds(0, CKA)], bufb0, sb0).wait()
            pltpu.sync_copy(bufb0, gb_hbm.at[pl.ds(base0, CKA)])
            pltpu.make_async_copy(xa_hbm.at[pl.ds(0, CKA)], bufg0, sg0).wait()
            pltpu.sync_copy(bufg0, g_hbm.at[pl.ds(base0, CKA)])

            @pl.when(j2 < NCHA)
            def _():
                issue0(j2)

            base1 = pl.multiple_of(w * EWP + j1 * CKA, CKA)
            pltpu.make_async_copy(a_hbm.at[pl.ds(0, CKA)], bufa1, sa1).wait()
            pltpu.sync_copy(bufa1, ga_hbm.at[pl.ds(base1, CKA)])
            pltpu.make_async_copy(b_hbm.at[pl.ds(0, CKA)], bufb1, sb1).wait()
            pltpu.sync_copy(bufb1, gb_hbm.at[pl.ds(base1, CKA)])
            pltpu.make_async_copy(xa_hbm.at[pl.ds(0, CKA)], bufg1, sg1).wait()
            pltpu.sync_copy(bufg1, g_hbm.at[pl.ds(base1, CKA)])
            return carry

        lax.fori_loop(0, NCHA // 2, chunk2, 0)

    f = pl.kernel(
        body,
        out_type=(jax.ShapeDtypeStruct((EP, D), jnp.float32),
                  jax.ShapeDtypeStruct((EP, D), jnp.float32),
                  jax.ShapeDtypeStruct((EP, D), jnp.float32)),
        mesh=_sc_mesh(),
        scratch_types=[
            pltpu.VMEM((CKA,), jnp.int32),
            pltpu.VMEM((CKA,), jnp.int32),
            pltpu.VMEM((CKA,), jnp.int32),
            pltpu.VMEM((CKA,), jnp.int32),
            pltpu.VMEM((CKA, D), jnp.float32),
            pltpu.VMEM((CKA, D), jnp.float32),
            pltpu.VMEM((CKA, D), jnp.float32),
            pltpu.VMEM((CKA, D), jnp.float32),
            pltpu.VMEM((CKA, D), jnp.float32),
            pltpu.VMEM((CKA, D), jnp.float32),
            pltpu.SemaphoreType.DMA,
            pltpu.SemaphoreType.DMA,
            pltpu.SemaphoreType.DMA,
            pltpu.SemaphoreType.DMA,
            pltpu.SemaphoreType.DMA,
            pltpu.SemaphoreType.DMA,
        ],
    )
    return f(A, B, XA, rowi, coli)


def _scatter_call(vals, rowi, zero):
    d = vals.shape[1]

    def body(val_hbm, rowi_hbm, zero_hbm, out_hbm,
             acc_sh, rbuf0, rbuf1, buf0, buf1, sem0, sem1):
        c = lax.axis_index("c")
        s = lax.axis_index("s")
        w = c * NS + s
        pltpu.sync_copy(zero_hbm, acc_sh.at[pl.ds(s * ZR, ZR)])
        plsc.subcore_barrier()

        base_w = pl.multiple_of(w * EWP, CK)
        pltpu.async_copy(val_hbm.at[pl.ds(base_w, CK)], buf0, sem0)

        def chunk2(g, carry):
            j0 = 2 * g
            j1 = j0 + 1
            j2 = j0 + 2
            base1 = pl.multiple_of(w * EWP + j1 * CK, CK)
            pltpu.async_copy(val_hbm.at[pl.ds(base1, CK)], buf1, sem1)
            pltpu.sync_copy(rowi_hbm.at[w, j0], rbuf0)
            pltpu.make_async_copy(val_hbm.at[pl.ds(0, CK)], buf0, sem0).wait()
            pltpu.sync_copy(buf0, acc_sh.at[rbuf0], add=True)

            @pl.when(j2 < NCH)
            def _():
                base2 = pl.multiple_of(w * EWP + j2 * CK, CK)
                pltpu.async_copy(val_hbm.at[pl.ds(base2, CK)], buf0, sem0)

            pltpu.sync_copy(rowi_hbm.at[w, j1], rbuf1)
            pltpu.make_async_copy(val_hbm.at[pl.ds(0, CK)], buf1, sem1).wait()
            pltpu.sync_copy(buf1, acc_sh.at[rbuf1], add=True)
            return carry

        lax.fori_loop(0, NCH // 2, chunk2, 0)
        plsc.subcore_barrier()
        pltpu.sync_copy(acc_sh.at[pl.ds(s * ZR, ZR)],
                        out_hbm.at[c, pl.ds(s * ZR, ZR)])

    f = pl.kernel(
        body,
        out_type=jax.ShapeDtypeStruct((NC, NP, d), jnp.float32),
        mesh=_sc_mesh(),
        scratch_types=[
            pltpu.VMEM_SHARED((NP, d), jnp.float32),
            pltpu.VMEM((CK,), jnp.int32),
            pltpu.VMEM((CK,), jnp.int32),
            pltpu.VMEM((CK, d), jnp.float32),
            pltpu.VMEM((CK, d), jnp.float32),
            pltpu.SemaphoreType.DMA,
            pltpu.SemaphoreType.DMA,
        ],
    )
    return f(vals, rowi, zero)


def _lin(x, W, b, act=None):
    n, din = x.shape
    dout = W.shape[0]
    bn = BN if n == N else BE

    def body(x_ref, w_ref, b_ref, o_ref):
        y = lax.dot_general(x_ref[...], w_ref[...], (((1,), (1,)), ((), ())),
                            preferred_element_type=jnp.float32)
        y = y + b_ref[...]
        if act == "relu":
            y = jnp.maximum(y, 0.0)
        elif act == "sigmoid":
            y = jax.nn.sigmoid(y)
        o_ref[...] = y

    return pl.pallas_call(
        body,
        grid=(n // bn,),
        in_specs=[
            pl.BlockSpec((bn, din), lambda i: (i, 0)),
            pl.BlockSpec((dout, din), lambda i: (0, 0)),
            pl.BlockSpec((1, dout), lambda i: (0, 0)),
        ],
        out_specs=pl.BlockSpec((bn, dout), lambda i: (i, 0)),
        out_shape=jax.ShapeDtypeStruct((n, dout), jnp.float32),
    )(x, W, b.reshape(1, dout))


def _combine(parts, fulls, rows, fn, n=N, d=D, n_out=1, bn=None):
    if bn is None:
        bn = BN if n == N else BE
    nf = len(fulls)
    nr = len(rows)

    def body(*refs):
        k = 0
        args = []
        if parts is not None:
            args.append(refs[0][...][0] + refs[1][...][0])
            k = 2
        for r in refs[k:k + nf + nr]:
            args.append(r[...])
        outs = refs[k + nf + nr:]
        res = fn(*args)
        if n_out == 1:
            res = (res,)
        for o, v in zip(outs, res):
            o[...] = v

    in_specs = []
    ops = []
    if parts is not None:
        in_specs.append(pl.BlockSpec((1, bn, d), lambda i: (0, i, 0)))
        in_specs.append(pl.BlockSpec((1, bn, d), lambda i: (1, i, 0)))
        ops += [parts, parts]
    for a in fulls:
        in_specs.append(pl.BlockSpec((bn, d), lambda i: (i, 0)))
        ops.append(a)
    for a in rows:
        in_specs.append(pl.BlockSpec((1, d), lambda i: (0, 0)))
        ops.append(a)
    out_shape = [jax.ShapeDtypeStruct((n, d), jnp.float32)] * n_out
    out_specs = [pl.BlockSpec((bn, d), lambda i: (i, 0))] * n_out
    res = pl.pallas_call(
        body, grid=(n // bn,), in_specs=in_specs, out_specs=out_specs,
        out_shape=out_shape)(*ops)
    return res[0] if n_out == 1 else res


def _stats_call(x, degb):
    def body(x_ref, d_ref, o_ref):
        i = pl.program_id(0)
        xb = x_ref[...]
        db = d_ref[...]
        blk = jnp.concatenate([
            jnp.sum(xb, axis=0, keepdims=True),
            jnp.sum(xb * xb, axis=0, keepdims=True),
            jnp.sum(db, axis=0, keepdims=True),
            jnp.sum(db * db, axis=0, keepdims=True),
            jnp.sum((db == 0.0).astype(jnp.float32), axis=0, keepdims=True),
            jnp.zeros((3, 128), jnp.float32),
        ], axis=0)

        @pl.when(i == 0)
        def _():
            o_ref[...] = blk

        @pl.when(i > 0)
        def _():
            o_ref[...] = o_ref[...] + blk

    return pl.pallas_call(
        body,
        grid=(N // BN,),
        in_specs=[pl.BlockSpec((BN, 128), lambda i: (i, 0)),
                  pl.BlockSpec((BN, 128), lambda i: (i, 0))],
        out_specs=pl.BlockSpec((8, 128), lambda i: (0, 0)),
        out_shape=jax.ShapeDtypeStruct((8, 128), jnp.float32),
    )(x, degb)


def _ctrl_call(h0p, W1p, b1, W2p, b2p):
    def body(h_ref, w1_ref, b1_ref, w2_ref, b2_ref, o_ref):
        r1 = lax.dot_general(h_ref[...], w1_ref[...], (((1,), (1,)), ((), ())),
                             preferred_element_type=jnp.float32) + b1_ref[...]
        r1 = jnp.maximum(r1, 0.0)
        o_ref[...] = lax.dot_general(
            r1, w2_ref[...], (((1,), (1,)), ((), ())),
            preferred_element_type=jnp.float32) + b2_ref[...]

    return pl.pallas_call(
        body,
        out_shape=jax.ShapeDtypeStruct((8, 128), jnp.float32),
    )(h0p, W1p, b1, W2p, b2p)


def _edge_score_call(GA, GB, W2, b2):
    def body(c_ref, a_ref, b_ref, w_ref, o_ref):
        r = jnp.maximum(a_ref[...] + b_ref[...], 0.0)
        o_ref[...] = lax.dot_general(
            r, w_ref[...], (((1,), (1,)), ((), ())),
            preferred_element_type=jnp.float32) + c_ref[0]

    return pl.pallas_call(
        body,
        grid=(EP // BE,),
        in_specs=[
            pl.BlockSpec(memory_space=pltpu.SMEM),
            pl.BlockSpec((BE, 128), lambda i: (i, 0)),
            pl.BlockSpec((BE, 128), lambda i: (i, 0)),
            pl.BlockSpec((8, 128), lambda i: (0, 0)),
        ],
        out_specs=pl.BlockSpec((BE, 8), lambda i: (i, 0)),
        out_shape=jax.ShapeDtypeStruct((EP, 8), jnp.float32),
    )(b2, GA, GB, W2)


def _redmax_call(a, mask):
    n, d = a.shape

    def body(a_ref, k_ref, o_ref):
        m = a_ref[...] * k_ref[...] - (1.0 - k_ref[...]) * 1e30
        o_ref[...] = jnp.max(m, axis=0, keepdims=True)

    return pl.pallas_call(
        body,
        out_shape=jax.ShapeDtypeStruct((1, d), jnp.float32))(a, mask)


def _redsumexp_call(a, mxr, mask):
    n, d = a.shape

    def body(a_ref, m_ref, k_ref, o_ref):
        o_ref[...] = jnp.sum(jnp.exp(a_ref[...] - m_ref[...]) * k_ref[...],
                             axis=0, keepdims=True)

    return pl.pallas_call(
        body,
        out_shape=jax.ShapeDtypeStruct((1, d), jnp.float32))(a, mxr, mask)


def _wmul_call(attn1, G):
    def body(a_ref, g_ref, o_ref):
        o_ref[...] = a_ref[...] * g_ref[...]

    return pl.pallas_call(
        body, grid=(EP // BE,),
        in_specs=[pl.BlockSpec((BE, 1), lambda i: (i, 0)),
                  pl.BlockSpec((BE, 128), lambda i: (i, 0))],
        out_specs=pl.BlockSpec((BE, 128), lambda i: (i, 0)),
        out_shape=jax.ShapeDtypeStruct((EP, 128), jnp.float32))(attn1, G)


def _row(v):
    return jnp.full((1, 128), 1.0, jnp.float32) * v


def kernel(edge_index, x, prev_emb, ctrl_W1, ctrl_b1, ctrl_W2, ctrl_b2,
           mode_bias, att_W1, att_b1, att_W2, att_b2, heat_kernels, time_W,
           time_b, scale_weights, sp_W, sp_b, tm_W, tm_b, at_W, at_b, df_W,
           df_b, hr_W, hr_b, out_W1, out_b1, out_W2, out_b2):
    row = edge_index[0]
    col = edge_index[1]
    kpad = jnp.arange(EWP, dtype=jnp.int32)
    padrow = jnp.broadcast_to(N + (kpad % (NP - N - 8)), (NW, EWP))
    padcol = jnp.broadcast_to((kpad * 797) % N, (NW, EWP))
    rowp = padrow.at[:, :EW].set(row.reshape(NW, EW))
    colp = padcol.at[:, :EW].set(col.reshape(NW, EW))
    rowi = rowp.reshape(NW, NCH, CK)
    coli = colp.reshape(NW, NCH, CK)
    rowia = rowp.reshape(NW, NCHA, CKA)
    colia = colp.reshape(NW, NCHA, CKA)
    rowiu = row.reshape(NW, NCHU, CKU)
    coliu = col.reshape(NW, NCHU, CKU)
    ke = jnp.arange(EP, dtype=jnp.int32)
    emask = (ke % EWP < EW).astype(jnp.float32)
    mask2d = emask.reshape(EP // 128, 128)
    zero = jnp.zeros((ZR, D), jnp.float32)

    Sdeg = _spmm_call(jnp.ones((N, D), jnp.float32), rowiu, coliu, zero)
    degb, degm, dinv = _combine(
        Sdeg, [], [],
        lambda p: (p, jnp.maximum(p, 1.0),
                   jnp.maximum(lax.rsqrt(p), 1e-8)),
        n_out=3)

    acc = _stats_call(x, degb)
    s_x = jnp.sum(acc[0])
    s_x2 = jnp.sum(acc[1])
    s_d = acc[2, 0]
    s_d2 = acc[3, 0]
    s_z = acc[4, 0]
    cnt = float(N * D)
    mean_x = s_x / cnt
    std_x = jnp.sqrt(jnp.maximum((s_x2 - cnt * mean_x * mean_x) / (cnt - 1.0),
                                 0.0))
    mean_d = s_d / N
    std_d = jnp.sqrt(jnp.maximum((s_d2 - N * mean_d * mean_d) / (N - 1.0),
                                 0.0))
    stats = jnp.stack([
        jnp.float32(N / 1000.0), jnp.float32(E / max(N, 1)), std_d, s_z / N,
        mean_x, std_x, jnp.float32(1.0), jnp.float32(E / (N * N)),
    ])
    quality = jnp.mean(prev_emb, axis=0)
    h0 = jnp.concatenate([stats, quality])
    h0p = jnp.zeros((8, 256), jnp.float32).at[0, :8 + H].set(h0)
    W1p = jnp.zeros((128, 256), jnp.float32).at[:, :8 + H].set(ctrl_W1)
    W2p = jnp.zeros((128, 128), jnp.float32).at[:5].set(ctrl_W2)
    b2p = jnp.zeros((1, 128), jnp.float32).at[0, :5].set(ctrl_b2)
    logits = _ctrl_call(h0p, W1p, ctrl_b1.reshape(1, 128), W2p, b2p)[0, :5]
    logits = logits + mode_bias
    u = jax.random.uniform(jax.random.key(42), (5,), dtype=jnp.float32)
    g = -jnp.log(-jnp.log(u + 1e-20) + 1e-20)
    probs = jax.nn.softmax((logits + g) / TAU)

    xs = x
    for i in range(L):
        y = _lin(xs, sp_W[i], sp_b[i])
        S = _spmm_call(y, rowiu, coliu, zero)
        xs = _combine(S, [degm], [],
                      lambda p, dm: jnp.maximum(p / dm, 0.0))

    xt = x
    tW = time_W[:, :H]
    for i in range(L):
        xt1 = _lin(xt, tm_W[i], tm_b[i])
        gate = _lin(xt1, tW, time_b, act="sigmoid")
        S = _spmm_call(xt1, rowiu, coliu, zero)
        xt = _combine(S, [xt1, gate, degm], [],
                      lambda p, z, gt, dm: jnp.maximum(
                          gt * z + (1.0 - gt) * (p / dm), 0.0))

    xa = x
    W1a = jnp.zeros((128, H), jnp.float32).at[:64].set(att_W1[:, :H])
    W1b = jnp.zeros((128, H), jnp.float32).at[:64].set(att_W1[:, H:])
    b1p = jnp.zeros((128,), jnp.float32).at[:64].set(att_b1)
    W2p = jnp.zeros((8, 128), jnp.float32).at[:1, :64].set(att_W2)
    for i in range(L):
        xa = _lin(xa, at_W[i], at_b[i])
        A = _lin(xa, W1a, b1p)
        B = _lin(xa, W1b, jnp.zeros((128,), jnp.float32))
        GA, GB, G = _att_gather_call(A, B, xa, rowia, colia)
        sc = _edge_score_call(GA, GB, W2p, att_b2)
        sc2d = sc[:, 0].reshape(EP // 128, 128)
        mx = jnp.max(_redmax_call(sc2d, mask2d))
        ssum = jnp.sum(_redsumexp_call(sc2d, _row(mx), mask2d))
        attn2d = _combine(None, [sc2d, mask2d], [_row(mx), _row(1.0 / ssum)],
                          lambda a, k, m, r: jnp.exp(a - m) * k * r,
                          n=EP // 128, d=128, bn=EP // 128)
        WG = _wmul_call(attn2d.reshape(EP, 1), G)
        S = _scatter_call(WG, rowi, zero)
        xa = _combine(S, [], [], lambda p: jnp.maximum(p, 0.0))

    xd = x
    for i in range(L):
        xd = _lin(xd, df_W[i], df_b[i])
        acc_d = jnp.zeros((N, D), jnp.float32)
        y = _combine(None, [xd, dinv], [], lambda a, b: a * b)
        for k in range(5):
            t = heat_kernels[k]
            S = _spmm_call(y, rowiu, coliu, zero)
            xd, y, acc_d = _combine(
                S, [xd, acc_d, dinv], [_row(t)],
                lambda p, xo, ao, dv, tr: (
                    (1.0 - tr) * xo + tr * (dv * p),
                    dv * ((1.0 - tr) * xo + tr * (dv * p)),
                    ao + (1.0 - tr) * xo + tr * (dv * p)),
                n_out=3)
        xd = _combine(None, [acc_d], [],
                      lambda a: jnp.maximum(a * (1.0 / 5.0), 0.0))

    w3 = jax.nn.softmax(scale_weights)
    xh = x
    for i in range(L):
        xh = _lin(xh, hr_W[i], hr_b[i])
        S1 = _spmm_call(xh, rowiu, coliu, zero)
        h1 = _combine(S1, [degm], [], lambda p, dm: p / dm)
        S2 = _spmm_call(h1, rowiu, coliu, zero)
        h2 = _combine(S2, [degm], [], lambda p, dm: p / dm)
        S3 = _spmm_call(h2, rowiu, coliu, zero)
        xh = _combine(S3, [h1, h2, degm], [_row(w3[0]), _row(w3[1]),
                                           _row(w3[2])],
                      lambda p, a, b, dm, w0, w1, w2: jnp.maximum(
                          w0 * a + w1 * b + w2 * (p / dm), 0.0))

    xout = _combine(None, [xs, xt, xa, xd, xh],
                    [_row(probs[0]), _row(probs[1]), _row(probs[2]),
                     _row(probs[3]), _row(probs[4])],
                    lambda a, b, c, d_, e, p0, p1, p2, p3, p4:
                    p0 * a + p1 * b + p2 * c + p3 * d_ + p4 * e)
    o1 = _lin(xout, out_W1, out_b1, act="relu")
    return _lin(o1, out_W2, out_b2)


# speedup vs baseline: 2.6762x; 1.2172x over previous
import functools

import jax
import jax.numpy as jnp
from jax import lax
from jax.experimental import pallas as pl
from jax.experimental.pallas import tpu as pltpu
from jax.experimental.pallas import tpu_sc as plsc

N = 10000
E = 320000
D = 128
H = 128
OUT = 128
L = 2
TAU = 0.5

NP = 10112
NC = 2
NS = 16
NW = NC * NS
EW = E // NW
CK = 128
EWP = 10240
NCH = EWP // CK
CKA = 128
NCHA = EWP // CKA
CKU = 80
NCHU = EW // CKU
EP = NW * EWP
ZR = NP // NS
PAD_ROW = NP - 8

BN = 1000
BE = 2048


def _sc_mesh():
    return plsc.VectorSubcoreMesh(
        core_axis_name="c", subcore_axis_name="s", num_cores=NC,
        num_subcores=NS)


def _spmm_call(y, rowi, coli, zero):
    d = y.shape[1]

    def body(y_hbm, rowi_hbm, coli_hbm, zero_hbm, out_hbm,
             acc_sh, coli_v, rbuf0, rbuf1, buf0, buf1, sem0, sem1):
        c = lax.axis_index("c")
        s = lax.axis_index("s")
        w = c * NS + s
        pltpu.sync_copy(zero_hbm, acc_sh.at[pl.ds(s * ZR, ZR)])
        pltpu.sync_copy(coli_hbm.at[w], coli_v)
        plsc.subcore_barrier()

        j_first = w * 0
        pltpu.async_copy(y_hbm.at[coli_v.at[j_first]], buf0, sem0)
        pltpu.sync_copy(rowi_hbm.at[w, j_first], rbuf0)
        pltpu.make_async_copy(y_hbm.at[pl.ds(0, CKU)], buf0, sem0).wait()
        pltpu.sync_copy(buf0, acc_sh.at[rbuf0], add=True)
        pltpu.async_copy(y_hbm.at[coli_v.at[j_first + 1]], buf0, sem0)

        def chunk2(g, carry):
            j0 = 2 * g + 1
            j1 = j0 + 1
            j2 = j0 + 2
            pltpu.async_copy(y_hbm.at[coli_v.at[j1]], buf1, sem1)
            pltpu.sync_copy(rowi_hbm.at[w, j0], rbuf0)
            pltpu.make_async_copy(y_hbm.at[pl.ds(0, CKU)], buf0, sem0).wait()
            pltpu.sync_copy(buf0, acc_sh.at[rbuf0], add=True)

            @pl.when(j2 < NCHU)
            def _():
                pltpu.async_copy(y_hbm.at[coli_v.at[j2]], buf0, sem0)

            pltpu.sync_copy(rowi_hbm.at[w, j1], rbuf1)
            pltpu.make_async_copy(y_hbm.at[pl.ds(0, CKU)], buf1, sem1).wait()
            pltpu.sync_copy(buf1, acc_sh.at[rbuf1], add=True)
            return carry

        lax.fori_loop(0, (NCHU - 1) // 2, chunk2, 0)
        plsc.subcore_barrier()
        pltpu.sync_copy(acc_sh.at[pl.ds(s * ZR, ZR)],
                        out_hbm.at[c, pl.ds(s * ZR, ZR)])

    f = pl.kernel(
        body,
        out_type=jax.ShapeDtypeStruct((NC, NP, d), jnp.float32),
        mesh=_sc_mesh(),
        scratch_types=[
            pltpu.VMEM_SHARED((NP, d), jnp.float32),
            pltpu.VMEM((NCHU, CKU), jnp.int32),
            pltpu.VMEM((CKU,), jnp.int32),
            pltpu.VMEM((CKU,), jnp.int32),
            pltpu.VMEM((CKU, d), jnp.float32),
            pltpu.VMEM((CKU, d), jnp.float32),
            pltpu.SemaphoreType.DMA,
            pltpu.SemaphoreType.DMA,
        ],
    )
    return f(y, rowi, coli, zero)


def _att_gather_call(A, B, XA, rowi, coli):
    def body(a_hbm, b_hbm, xa_hbm, rowi_hbm, coli_hbm,
             ga_hbm, gb_hbm, g_hbm,
             rb0, rb1, cb0, cb1, bufa0, bufb0, bufg0, bufa1, bufb1, bufg1,
             sa0, sb0, sg0, sa1, sb1, sg1):
        c = lax.axis_index("c")
        s = lax.axis_index("s")
        w = c * NS + s

        def issue0(j):
            pltpu.sync_copy(rowi_hbm.at[w, j], rb0)
            pltpu.sync_copy(coli_hbm.at[w, j], cb0)
            pltpu.async_copy(a_hbm.at[rb0], bufa0, sa0)
            pltpu.async_copy(b_hbm.at[cb0], bufb0, sb0)
            pltpu.async_copy(xa_hbm.at[cb0], bufg0, sg0)

        def issue1(j):
            pltpu.sync_copy(rowi_hbm.at[w, j], rb1)
            pltpu.sync_copy(coli_hbm.at[w, j], cb1)
            pltpu.async_copy(a_hbm.at[rb1], bufa1, sa1)
            pltpu.async_copy(b_hbm.at[cb1], bufb1, sb1)
            pltpu.async_copy(xa_hbm.at[cb1], bufg1, sg1)

        j_first = w * 0
        issue0(j_first)

        def chunk2(g, carry):
            j0 = 2 * g
            j1 = j0 + 1
            j2 = j0 + 2
            issue1(j1)
            base0 = pl.multiple_of(w * EWP + j0 * CKA, CKA)
            pltpu.make_async_copy(a_hbm.at[pl.ds(0, CKA)], bufa0, sa0).wait()
            pltpu.sync_copy(bufa0, ga_hbm.at[pl.ds(base0, CKA)])
            pltpu.make_async_copy(b_hbm.at[pl.ds(0, CKA)], bufb0, sb0).wait()
            pltpu.sync_copy(bufb0, gb_hbm.at[pl.ds(base0, CKA)])
            pltpu.make_async_copy(xa_hbm.at[pl.ds(0, CKA)], bufg0, sg0).wait()
            pltpu.sync_copy(bufg0, g_hbm.at[pl.ds(base0, CKA)])

            @pl.when(j2 < NCHA)
            def _():
                issue0(j2)

            base1 = pl.multiple_of(w * EWP + j1 * CKA, CKA)
            pltpu.make_async_copy(a_hbm.at[pl.ds(0, CKA)], bufa1, sa1).wait()
            pltpu.sync_copy(bufa1, ga_hbm.at[pl.ds(base1, CKA)])
            pltpu.make_async_copy(b_hbm.at[pl.ds(0, CKA)], bufb1, sb1).wait()
            pltpu.sync_copy(bufb1, gb_hbm.at[pl.ds(base1, CKA)])
            pltpu.make_async_copy(xa_hbm.at[pl.ds(0, CKA)], bufg1, sg1).wait()
            pltpu.sync_copy(bufg1, g_hbm.at[pl.ds(base1, CKA)])
            return carry

        lax.fori_loop(0, NCHA // 2, chunk2, 0)

    f = pl.kernel(
        body,
        out_type=(jax.ShapeDtypeStruct((EP, D), jnp.float32),
                  jax.ShapeDtypeStruct((EP, D), jnp.float32),
                  jax.ShapeDtypeStruct((EP, D), jnp.float32)),
        mesh=_sc_mesh(),
        scratch_types=[
            pltpu.VMEM((CKA,), jnp.int32),
            pltpu.VMEM((CKA,), jnp.int32),
            pltpu.VMEM((CKA,), jnp.int32),
            pltpu.VMEM((CKA,), jnp.int32),
            pltpu.VMEM((CKA, D), jnp.float32),
            pltpu.VMEM((CKA, D), jnp.float32),
            pltpu.VMEM((CKA, D), jnp.float32),
            pltpu.VMEM((CKA, D), jnp.float32),
            pltpu.VMEM((CKA, D), jnp.float32),
            pltpu.VMEM((CKA, D), jnp.float32),
            pltpu.SemaphoreType.DMA,
            pltpu.SemaphoreType.DMA,
            pltpu.SemaphoreType.DMA,
            pltpu.SemaphoreType.DMA,
            pltpu.SemaphoreType.DMA,
            pltpu.SemaphoreType.DMA,
        ],
    )
    return f(A, B, XA, rowi, coli)


def _scatter_call(vals, rowi, zero):
    d = vals.shape[1]

    def body(val_hbm, rowi_hbm, zero_hbm, out_hbm,
             acc_sh, rbuf0, rbuf1, buf0, buf1, sem0, sem1):
        c = lax.axis_index("c")
        s = lax.axis_index("s")
        w = c * NS + s
        pltpu.sync_copy(zero_hbm, acc_sh.at[pl.ds(s * ZR, ZR)])
        plsc.subcore_barrier()

        base_w = pl.multiple_of(w * EWP, CK)
        pltpu.async_copy(val_hbm.at[pl.ds(base_w, CK)], buf0, sem0)

        def chunk2(g, carry):
            j0 = 2 * g
            j1 = j0 + 1
            j2 = j0 + 2
            base1 = pl.multiple_of(w * EWP + j1 * CK, CK)
            pltpu.async_copy(val_hbm.at[pl.ds(base1, CK)], buf1, sem1)
            pltpu.sync_copy(rowi_hbm.at[w, j0], rbuf0)
            pltpu.make_async_copy(val_hbm.at[pl.ds(0, CK)], buf0, sem0).wait()
            pltpu.sync_copy(buf0, acc_sh.at[rbuf0], add=True)

            @pl.when(j2 < NCH)
            def _():
                base2 = pl.multiple_of(w * EWP + j2 * CK, CK)
                pltpu.async_copy(val_hbm.at[pl.ds(base2, CK)], buf0, sem0)

            pltpu.sync_copy(rowi_hbm.at[w, j1], rbuf1)
            pltpu.make_async_copy(val_hbm.at[pl.ds(0, CK)], buf1, sem1).wait()
            pltpu.sync_copy(buf1, acc_sh.at[rbuf1], add=True)
            return carry

        lax.fori_loop(0, NCH // 2, chunk2, 0)
        plsc.subcore_barrier()
        pltpu.sync_copy(acc_sh.at[pl.ds(s * ZR, ZR)],
                        out_hbm.at[c, pl.ds(s * ZR, ZR)])

    f = pl.kernel(
        body,
        out_type=jax.ShapeDtypeStruct((NC, NP, d), jnp.float32),
        mesh=_sc_mesh(),
        scratch_types=[
            pltpu.VMEM_SHARED((NP, d), jnp.float32),
            pltpu.VMEM((CK,), jnp.int32),
            pltpu.VMEM((CK,), jnp.int32),
            pltpu.VMEM((CK, d), jnp.float32),
            pltpu.VMEM((CK, d), jnp.float32),
            pltpu.SemaphoreType.DMA,
            pltpu.SemaphoreType.DMA,
        ],
    )
    return f(vals, rowi, zero)


def _lin(x, W, b, act=None):
    n, din = x.shape
    dout = W.shape[0]
    bn = BN if n == N else BE

    def body(x_ref, w_ref, b_ref, o_ref):
        y = lax.dot_general(x_ref[...], w_ref[...], (((1,), (1,)), ((), ())),
                            preferred_element_type=jnp.float32)
        y = y + b_ref[...]
        if act == "relu":
            y = jnp.maximum(y, 0.0)
        elif act == "sigmoid":
            y = jax.nn.sigmoid(y)
        o_ref[...] = y

    return pl.pallas_call(
        body,
        grid=(n // bn,),
        in_specs=[
            pl.BlockSpec((bn, din), lambda i: (i, 0)),
            pl.BlockSpec((dout, din), lambda i: (0, 0)),
            pl.BlockSpec((1, dout), lambda i: (0, 0)),
        ],
        out_specs=pl.BlockSpec((bn, dout), lambda i: (i, 0)),
        out_shape=jax.ShapeDtypeStruct((n, dout), jnp.float32),
    )(x, W, b.reshape(1, dout))


def _combine(parts, fulls, rows, fn, n=N, d=D, n_out=1, bn=None):
    if bn is None:
        bn = BN if n == N else BE
    nf = len(fulls)
    nr = len(rows)

    def body(*refs):
        k = 0
        args = []
        if parts is not None:
            args.append(refs[0][...][0] + refs[1][...][0])
            k = 2
        for r in refs[k:k + nf + nr]:
            args.append(r[...])
        outs = refs[k + nf + nr:]
        res = fn(*args)
        if n_out == 1:
            res = (res,)
        for o, v in zip(outs, res):
            o[...] = v

    in_specs = []
    ops = []
    if parts is not None:
        in_specs.append(pl.BlockSpec((1, bn, d), lambda i: (0, i, 0)))
        in_specs.append(pl.BlockSpec((1, bn, d), lambda i: (1, i, 0)))
        ops += [parts, parts]
    for a in fulls:
        in_specs.append(pl.BlockSpec((bn, d), lambda i: (i, 0)))
        ops.append(a)
    for a in rows:
        in_specs.append(pl.BlockSpec((1, d), lambda i: (0, 0)))
        ops.append(a)
    out_shape = [jax.ShapeDtypeStruct((n, d), jnp.float32)] * n_out
    out_specs = [pl.BlockSpec((bn, d), lambda i: (i, 0))] * n_out
    res = pl.pallas_call(
        body, grid=(n // bn,), in_specs=in_specs, out_specs=out_specs,
        out_shape=out_shape)(*ops)
    return res[0] if n_out == 1 else res


def _stats_call(x, degb):
    def body(x_ref, d_ref, o_ref):
        i = pl.program_id(0)
        xb = x_ref[...]
        db = d_ref[...]
        blk = jnp.concatenate([
            jnp.sum(xb, axis=0, keepdims=True),
            jnp.sum(xb * xb, axis=0, keepdims=True),
            jnp.sum(db, axis=0, keepdims=True),
            jnp.sum(db * db, axis=0, keepdims=True),
            jnp.sum((db == 0.0).astype(jnp.float32), axis=0, keepdims=True),
            jnp.zeros((3, 128), jnp.float32),
        ], axis=0)

        @pl.when(i == 0)
        def _():
            o_ref[...] = blk

        @pl.when(i > 0)
        def _():
            o_ref[...] = o_ref[...] + blk

    return pl.pallas_call(
        body,
        grid=(N // BN,),
        in_specs=[pl.BlockSpec((BN, 128), lambda i: (i, 0)),
                  pl.BlockSpec((BN, 128), lambda i: (i, 0))],
        out_specs=pl.BlockSpec((8, 128), lambda i: (0, 0)),
        out_shape=jax.ShapeDtypeStruct((8, 128), jnp.float32),
    )(x, degb)


def _ctrl_call(h0p, W1p, b1, W2p, b2p):
    def body(h_ref, w1_ref, b1_ref, w2_ref, b2_ref, o_ref):
        r1 = lax.dot_general(h_ref[...], w1_ref[...], (((1,), (1,)), ((), ())),
                             preferred_element_type=jnp.float32) + b1_ref[...]
        r1 = jnp.maximum(r1, 0.0)
        o_ref[...] = lax.dot_general(
            r1, w2_ref[...], (((1,), (1,)), ((), ())),
            preferred_element_type=jnp.float32) + b2_ref[...]

    return pl.pallas_call(
        body,
        out_shape=jax.ShapeDtypeStruct((8, 128), jnp.float32),
    )(h0p, W1p, b1, W2p, b2p)


def _edge_score_call(GA, GB, W2, b2):
    def body(c_ref, a_ref, b_ref, w_ref, o_ref):
        r = jnp.maximum(a_ref[...] + b_ref[...], 0.0)
        o_ref[...] = lax.dot_general(
            r, w_ref[...], (((1,), (1,)), ((), ())),
            preferred_element_type=jnp.float32) + c_ref[0]

    return pl.pallas_call(
        body,
        grid=(EP // BE,),
        in_specs=[
            pl.BlockSpec(memory_space=pltpu.SMEM),
            pl.BlockSpec((BE, 128), lambda i: (i, 0)),
            pl.BlockSpec((BE, 128), lambda i: (i, 0)),
            pl.BlockSpec((8, 128), lambda i: (0, 0)),
        ],
        out_specs=pl.BlockSpec((BE, 8), lambda i: (i, 0)),
        out_shape=jax.ShapeDtypeStruct((EP, 8), jnp.float32),
    )(b2, GA, GB, W2)


def _redmax_call(a, mask):
    n, d = a.shape

    def body(a_ref, k_ref, o_ref):
        m = a_ref[...] * k_ref[...] - (1.0 - k_ref[...]) * 1e30
        o_ref[...] = jnp.max(m, axis=0, keepdims=True)

    return pl.pallas_call(
        body,
        out_shape=jax.ShapeDtypeStruct((1, d), jnp.float32))(a, mask)


def _redsumexp_call(a, mxr, mask):
    n, d = a.shape

    def body(a_ref, m_ref, k_ref, o_ref):
        o_ref[...] = jnp.sum(jnp.exp(a_ref[...] - m_ref[...]) * k_ref[...],
                             axis=0, keepdims=True)

    return pl.pallas_call(
        body,
        out_shape=jax.ShapeDtypeStruct((1, d), jnp.float32))(a, mxr, mask)


def _wmul_call(attn1, G):
    def body(a_ref, g_ref, o_ref):
        o_ref[...] = a_ref[...] * g_ref[...]

    return pl.pallas_call(
        body, grid=(EP // BE,),
        in_specs=[pl.BlockSpec((BE, 1), lambda i: (i, 0)),
                  pl.BlockSpec((BE, 128), lambda i: (i, 0))],
        out_specs=pl.BlockSpec((BE, 128), lambda i: (i, 0)),
        out_shape=jax.ShapeDtypeStruct((EP, 128), jnp.float32))(attn1, G)


def _row(v):
    return jnp.full((1, 128), 1.0, jnp.float32) * v


def kernel(edge_index, x, prev_emb, ctrl_W1, ctrl_b1, ctrl_W2, ctrl_b2,
           mode_bias, att_W1, att_b1, att_W2, att_b2, heat_kernels, time_W,
           time_b, scale_weights, sp_W, sp_b, tm_W, tm_b, at_W, at_b, df_W,
           df_b, hr_W, hr_b, out_W1, out_b1, out_W2, out_b2):
    row = edge_index[0]
    col = edge_index[1]
    kpad = jnp.arange(EWP, dtype=jnp.int32)
    padrow = jnp.broadcast_to(N + (kpad % (NP - N - 8)), (NW, EWP))
    padcol = jnp.broadcast_to((kpad * 797) % N, (NW, EWP))
    rowp = padrow.at[:, :EW].set(row.reshape(NW, EW))
    colp = padcol.at[:, :EW].set(col.reshape(NW, EW))
    rowi = rowp.reshape(NW, NCH, CK)
    coli = colp.reshape(NW, NCH, CK)
    rowia = rowp.reshape(NW, NCHA, CKA)
    colia = colp.reshape(NW, NCHA, CKA)
    rowiu = row.reshape(NW, NCHU, CKU)
    coliu = col.reshape(NW, NCHU, CKU)
    ke = jnp.arange(EP, dtype=jnp.int32)
    emask = (ke % EWP < EW).astype(jnp.float32)
    mask2d = emask.reshape(EP // 128, 128)
    zero = jnp.zeros((ZR, D), jnp.float32)

    Sdeg = _spmm_call(jnp.ones((N, D), jnp.float32), rowiu, coliu, zero)
    degb, degm, dinv = _combine(
        Sdeg, [], [],
        lambda p: (p, jnp.maximum(p, 1.0),
                   jnp.maximum(lax.rsqrt(p), 1e-8)),
        n_out=3)

    acc = _stats_call(x, degb)
    s_x = jnp.sum(acc[0])
    s_x2 = jnp.sum(acc[1])
    s_d = acc[2, 0]
    s_d2 = acc[3, 0]
    s_z = acc[4, 0]
    cnt = float(N * D)
    mean_x = s_x / cnt
    std_x = jnp.sqrt(jnp.maximum((s_x2 - cnt * mean_x * mean_x) / (cnt - 1.0),
                                 0.0))
    mean_d = s_d / N
    std_d = jnp.sqrt(jnp.maximum((s_d2 - N * mean_d * mean_d) / (N - 1.0),
                                 0.0))
    stats = jnp.stack([
        jnp.float32(N / 1000.0), jnp.float32(E / max(N, 1)), std_d, s_z / N,
        mean_x, std_x, jnp.float32(1.0), jnp.float32(E / (N * N)),
    ])
    quality = jnp.mean(prev_emb, axis=0)
    h0 = jnp.concatenate([stats, quality])
    h0p = jnp.zeros((8, 256), jnp.float32).at[0, :8 + H].set(h0)
    W1p = jnp.zeros((128, 256), jnp.float32).at[:, :8 + H].set(ctrl_W1)
    W2p = jnp.zeros((128, 128), jnp.float32).at[:5].set(ctrl_W2)
    b2p = jnp.zeros((1, 128), jnp.float32).at[0, :5].set(ctrl_b2)
    logits = _ctrl_call(h0p, W1p, ctrl_b1.reshape(1, 128), W2p, b2p)[0, :5]
    logits = logits + mode_bias
    u = jax.random.uniform(jax.random.key(42), (5,), dtype=jnp.float32)
    g = -jnp.log(-jnp.log(u + 1e-20) + 1e-20)
    probs = jax.nn.softmax((logits + g) / TAU)

    xs = x
    for i in range(L):
        y = _lin(xs, sp_W[i], sp_b[i])
        S = _spmm_call(y, rowiu, coliu, zero)
        xs = _combine(S, [degm], [],
                      lambda p, dm: jnp.maximum(p / dm, 0.0))

    xt = x
    tW = time_W[:, :H]
    for i in range(L):
        xt1 = _lin(xt, tm_W[i], tm_b[i])
        gate = _lin(xt1, tW, time_b, act="sigmoid")
        S = _spmm_call(xt1, rowiu, coliu, zero)
        xt = _combine(S, [xt1, gate, degm], [],
                      lambda p, z, gt, dm: jnp.maximum(
                          gt * z + (1.0 - gt) * (p / dm), 0.0))

    xa = x
    W1a = jnp.zeros((128, H), jnp.float32).at[:64].set(att_W1[:, :H])
    W1b = jnp.zeros((128, H), jnp.float32).at[:64].set(att_W1[:, H:])
    b1p = jnp.zeros((128,), jnp.float32).at[:64].set(att_b1)
    W2p = jnp.zeros((8, 128), jnp.float32).at[:1, :64].set(att_W2)
    for i in range(L):
        xa = _lin(xa, at_W[i], at_b[i])
        A = _lin(xa, W1a, b1p)
        B = _lin(xa, W1b, jnp.zeros((128,), jnp.float32))
        GA, GB, G = _att_gather_call(A, B, xa, rowia, colia)
        sc = _edge_score_call(GA, GB, W2p, att_b2)
        sc2d = sc[:, 0].reshape(EP // 128, 128)
        mx = jnp.max(_redmax_call(sc2d, mask2d))
        ssum = jnp.sum(_redsumexp_call(sc2d, _row(mx), mask2d))
        attn2d = _combine(None, [sc2d, mask2d], [_row(mx), _row(1.0 / ssum)],
                          lambda a, k, m, r: jnp.exp(a - m) * k * r,
                          n=EP // 128, d=128, bn=EP // 128)
        WG = _wmul_call(attn2d.reshape(EP, 1), G)
        S = _scatter_call(WG, rowi, zero)
        xa = _combine(S, [], [], lambda p: jnp.maximum(p, 0.0))

    xd = x
    for i in range(L):
        xd = _lin(xd, df_W[i], df_b[i])
        acc_d = jnp.zeros((N, D), jnp.float32)
        y = _combine(None, [xd, dinv], [], lambda a, b: a * b)
        for k in range(5):
            t = heat_kernels[k]
            S = _spmm_call(y, rowiu, coliu, zero)
            xd, y, acc_d = _combine(
                S, [xd, acc_d, dinv], [_row(t)],
                lambda p, xo, ao, dv, tr: (
                    (1.0 - tr) * xo + tr * (dv * p),
                    dv * ((1.0 - tr) * xo + tr * (dv * p)),
                    ao + (1.0 - tr) * xo + tr * (dv * p)),
                n_out=3)
        xd = _combine(None, [acc_d], [],
                      lambda a: jnp.maximum(a * (1.0 / 5.0), 0.0))

    w3 = jax.nn.softmax(scale_weights)
    xh = x
    for i in range(L):
        xh = _lin(xh, hr_W[i], hr_b[i])
        S1 = _spmm_call(xh, rowiu, coliu, zero)
        h1 = _combine(S1, [degm], [], lambda p, dm: p / dm)
        S2 = _spmm_call(h1, rowiu, coliu, zero)
        h2 = _combine(S2, [degm], [], lambda p, dm: p / dm)
        S3 = _spmm_call(h2, rowiu, coliu, zero)
        xh = _combine(S3, [h1, h2, degm], [_row(w3[0]), _row(w3[1]),
                                           _row(w3[2])],
                      lambda p, a, b, dm, w0, w1, w2: jnp.maximum(
                          w0 * a + w1 * b + w2 * (p / dm), 0.0))

    xout = _combine(None, [xs, xt, xa, xd, xh],
                    [_row(probs[0]), _row(probs[1]), _row(probs[2]),
                     _row(probs[3]), _row(probs[4])],
                    lambda a, b, c, d_, e, p0, p1, p2, p3, p4:
                    p0 * a + p1 * b + p2 * c + p3 * d_ + p4 * e)
    o1 = _lin(xout, out_W1, out_b1, act="relu")
    return _lin(o1, out_W2, out_b2)


# speedup vs baseline: 2.6918x; 1.0058x over previous
import functools

import jax
import jax.numpy as jnp
from jax import lax
from jax.experimental import pallas as pl
from jax.experimental.pallas import tpu as pltpu
from jax.experimental.pallas import tpu_sc as plsc

N = 10000
E = 320000
D = 128
H = 128
OUT = 128
L = 2
TAU = 0.5

NP = 10112
NC = 2
NS = 16
NW = NC * NS
EW = E // NW
CK = 128
EWP = 10240
NCH = EWP // CK
CKA = 128
NCHA = EWP // CKA
CKU = 80
NCHU = EW // CKU
EP = NW * EWP
ZR = NP // NS
PAD_ROW = NP - 8

BN = 1000
BE = 2048


def _sc_mesh():
    return plsc.VectorSubcoreMesh(
        core_axis_name="c", subcore_axis_name="s", num_cores=NC,
        num_subcores=NS)


def _spmm_call(y, rowi, coli, zero):
    d = y.shape[1]

    def body(y_hbm, rowi_hbm, coli_hbm, zero_hbm, out_hbm,
             acc_sh, coli_v, rbuf0, rbuf1, buf0, buf1, sem0, sem1):
        c = lax.axis_index("c")
        s = lax.axis_index("s")
        w = c * NS + s
        pltpu.sync_copy(zero_hbm, acc_sh.at[pl.ds(s * ZR, ZR)])
        pltpu.sync_copy(coli_hbm.at[w], coli_v)
        plsc.subcore_barrier()

        j_first = w * 0
        pltpu.async_copy(y_hbm.at[coli_v.at[j_first]], buf0, sem0)
        pltpu.sync_copy(rowi_hbm.at[w, j_first], rbuf0)
        pltpu.make_async_copy(y_hbm.at[pl.ds(0, CKU)], buf0, sem0).wait()
        pltpu.sync_copy(buf0, acc_sh.at[rbuf0], add=True)
        pltpu.async_copy(y_hbm.at[coli_v.at[j_first + 1]], buf0, sem0)

        def chunk2(g, carry):
            j0 = 2 * g + 1
            j1 = j0 + 1
            j2 = j0 + 2
            pltpu.async_copy(y_hbm.at[coli_v.at[j1]], buf1, sem1)
            pltpu.sync_copy(rowi_hbm.at[w, j0], rbuf0)
            pltpu.make_async_copy(y_hbm.at[pl.ds(0, CKU)], buf0, sem0).wait()
            pltpu.sync_copy(buf0, acc_sh.at[rbuf0], add=True)

            @pl.when(j2 < NCHU)
            def _():
                pltpu.async_copy(y_hbm.at[coli_v.at[j2]], buf0, sem0)

            pltpu.sync_copy(rowi_hbm.at[w, j1], rbuf1)
            pltpu.make_async_copy(y_hbm.at[pl.ds(0, CKU)], buf1, sem1).wait()
            pltpu.sync_copy(buf1, acc_sh.at[rbuf1], add=True)
            return carry

        lax.fori_loop(0, (NCHU - 1) // 2, chunk2, 0)
        plsc.subcore_barrier()
        pltpu.sync_copy(acc_sh.at[pl.ds(s * ZR, ZR)],
                        out_hbm.at[c, pl.ds(s * ZR, ZR)])

    f = pl.kernel(
        body,
        out_type=jax.ShapeDtypeStruct((NC, NP, d), jnp.float32),
        mesh=_sc_mesh(),
        scratch_types=[
            pltpu.VMEM_SHARED((NP, d), jnp.float32),
            pltpu.VMEM((NCHU, CKU), jnp.int32),
            pltpu.VMEM((CKU,), jnp.int32),
            pltpu.VMEM((CKU,), jnp.int32),
            pltpu.VMEM((CKU, d), jnp.float32),
            pltpu.VMEM((CKU, d), jnp.float32),
            pltpu.SemaphoreType.DMA,
            pltpu.SemaphoreType.DMA,
        ],
    )
    return f(y, rowi, coli, zero)


def _att_gather_call(A, B, XA, rowi, coli):
    def body(a_hbm, b_hbm, xa_hbm, rowi_hbm, coli_hbm,
             ga_hbm, gb_hbm, g_hbm,
             rowi_v, coli_v, bufa0, bufb0, bufg0, bufa1, bufb1, bufg1,
             sa0, sb0, sg0, sa1, sb1, sg1):
        c = lax.axis_index("c")
        s = lax.axis_index("s")
        w = c * NS + s
        pltpu.sync_copy(rowi_hbm.at[w], rowi_v)
        pltpu.sync_copy(coli_hbm.at[w], coli_v)

        def issue0(j):
            pltpu.async_copy(a_hbm.at[rowi_v.at[j]], bufa0, sa0)
            pltpu.async_copy(b_hbm.at[coli_v.at[j]], bufb0, sb0)
            pltpu.async_copy(xa_hbm.at[coli_v.at[j]], bufg0, sg0)

        def issue1(j):
            pltpu.async_copy(a_hbm.at[rowi_v.at[j]], bufa1, sa1)
            pltpu.async_copy(b_hbm.at[coli_v.at[j]], bufb1, sb1)
            pltpu.async_copy(xa_hbm.at[coli_v.at[j]], bufg1, sg1)

        j_first = w * 0
        issue0(j_first)

        def chunk2(g, carry):
            j0 = 2 * g
            j1 = j0 + 1
            j2 = j0 + 2
            issue1(j1)
            base0 = pl.multiple_of(w * EWP + j0 * CKA, CKA)
            pltpu.make_async_copy(a_hbm.at[pl.ds(0, CKA)], bufa0, sa0).wait()
            pltpu.sync_copy(bufa0, ga_hbm.at[pl.ds(base0, CKA)])
            pltpu.make_async_copy(b_hbm.at[pl.ds(0, CKA)], bufb0, sb0).wait()
            pltpu.sync_copy(bufb0, gb_hbm.at[pl.ds(base0, CKA)])
            pltpu.make_async_copy(xa_hbm.at[pl.ds(0, CKA)], bufg0, sg0).wait()
            pltpu.sync_copy(bufg0, g_hbm.at[pl.ds(base0, CKA)])

            @pl.when(j2 < NCHA)
            def _():
                issue0(j2)

            base1 = pl.multiple_of(w * EWP + j1 * CKA, CKA)
            pltpu.make_async_copy(a_hbm.at[pl.ds(0, CKA)], bufa1, sa1).wait()
            pltpu.sync_copy(bufa1, ga_hbm.at[pl.ds(base1, CKA)])
            pltpu.make_async_copy(b_hbm.at[pl.ds(0, CKA)], bufb1, sb1).wait()
            pltpu.sync_copy(bufb1, gb_hbm.at[pl.ds(base1, CKA)])
            pltpu.make_async_copy(xa_hbm.at[pl.ds(0, CKA)], bufg1, sg1).wait()
            pltpu.sync_copy(bufg1, g_hbm.at[pl.ds(base1, CKA)])
            return carry

        lax.fori_loop(0, NCHA // 2, chunk2, 0)

    f = pl.kernel(
        body,
        out_type=(jax.ShapeDtypeStruct((EP, D), jnp.float32),
                  jax.ShapeDtypeStruct((EP, D), jnp.float32),
                  jax.ShapeDtypeStruct((EP, D), jnp.float32)),
        mesh=_sc_mesh(),
        scratch_types=[
            pltpu.VMEM((NCHA, CKA), jnp.int32),
            pltpu.VMEM((NCHA, CKA), jnp.int32),
            pltpu.VMEM((CKA, D), jnp.float32),
            pltpu.VMEM((CKA, D), jnp.float32),
            pltpu.VMEM((CKA, D), jnp.float32),
            pltpu.VMEM((CKA, D), jnp.float32),
            pltpu.VMEM((CKA, D), jnp.float32),
            pltpu.VMEM((CKA, D), jnp.float32),
            pltpu.SemaphoreType.DMA,
            pltpu.SemaphoreType.DMA,
            pltpu.SemaphoreType.DMA,
            pltpu.SemaphoreType.DMA,
            pltpu.SemaphoreType.DMA,
            pltpu.SemaphoreType.DMA,
        ],
    )
    return f(A, B, XA, rowi, coli)


def _scatter_call(vals, rowi, zero):
    d = vals.shape[1]

    def body(val_hbm, rowi_hbm, zero_hbm, out_hbm,
             acc_sh, rbuf0, rbuf1, buf0, buf1, sem0, sem1):
        c = lax.axis_index("c")
        s = lax.axis_index("s")
        w = c * NS + s
        pltpu.sync_copy(zero_hbm, acc_sh.at[pl.ds(s * ZR, ZR)])
        plsc.subcore_barrier()

        base_w = pl.multiple_of(w * EWP, CK)
        pltpu.async_copy(val_hbm.at[pl.ds(base_w, CK)], buf0, sem0)

        def chunk2(g, carry):
            j0 = 2 * g
            j1 = j0 + 1
            j2 = j0 + 2
            base1 = pl.multiple_of(w * EWP + j1 * CK, CK)
            pltpu.async_copy(val_hbm.at[pl.ds(base1, CK)], buf1, sem1)
            pltpu.sync_copy(rowi_hbm.at[w, j0], rbuf0)
            pltpu.make_async_copy(val_hbm.at[pl.ds(0, CK)], buf0, sem0).wait()
            pltpu.sync_copy(buf0, acc_sh.at[rbuf0], add=True)

            @pl.when(j2 < NCH)
            def _():
                base2 = pl.multiple_of(w * EWP + j2 * CK, CK)
                pltpu.async_copy(val_hbm.at[pl.ds(base2, CK)], buf0, sem0)

            pltpu.sync_copy(rowi_hbm.at[w, j1], rbuf1)
            pltpu.make_async_copy(val_hbm.at[pl.ds(0, CK)], buf1, sem1).wait()
            pltpu.sync_copy(buf1, acc_sh.at[rbuf1], add=True)
            return carry

        lax.fori_loop(0, NCH // 2, chunk2, 0)
        plsc.subcore_barrier()
        pltpu.sync_copy(acc_sh.at[pl.ds(s * ZR, ZR)],
                        out_hbm.at[c, pl.ds(s * ZR, ZR)])

    f = pl.kernel(
        body,
        out_type=jax.ShapeDtypeStruct((NC, NP, d), jnp.float32),
        mesh=_sc_mesh(),
        scratch_types=[
            pltpu.VMEM_SHARED((NP, d), jnp.float32),
            pltpu.VMEM((CK,), jnp.int32),
            pltpu.VMEM((CK,), jnp.int32),
            pltpu.VMEM((CK, d), jnp.float32),
            pltpu.VMEM((CK, d), jnp.float32),
            pltpu.SemaphoreType.DMA,
            pltpu.SemaphoreType.DMA,
        ],
    )
    return f(vals, rowi, zero)


def _lin(x, W, b, act=None):
    n, din = x.shape
    dout = W.shape[0]
    bn = BN if n == N else BE

    def body(x_ref, w_ref, b_ref, o_ref):
        y = lax.dot_general(x_ref[...], w_ref[...], (((1,), (1,)), ((), ())),
                            preferred_element_type=jnp.float32)
        y = y + b_ref[...]
        if act == "relu":
            y = jnp.maximum(y, 0.0)
        elif act == "sigmoid":
            y = jax.nn.sigmoid(y)
        o_ref[...] = y

    return pl.pallas_call(
        body,
        grid=(n // bn,),
        in_specs=[
            pl.BlockSpec((bn, din), lambda i: (i, 0)),
            pl.BlockSpec((dout, din), lambda i: (0, 0)),
            pl.BlockSpec((1, dout), lambda i: (0, 0)),
        ],
        out_specs=pl.BlockSpec((bn, dout), lambda i: (i, 0)),
        out_shape=jax.ShapeDtypeStruct((n, dout), jnp.float32),
    )(x, W, b.reshape(1, dout))


def _combine(parts, fulls, rows, fn, n=N, d=D, n_out=1, bn=None):
    if bn is None:
        bn = BN if n == N else BE
    nf = len(fulls)
    nr = len(rows)

    def body(*refs):
        k = 0
        args = []
        if parts is not None:
            args.append(refs[0][...][0] + refs[1][...][0])
            k = 2
        for r in refs[k:k + nf + nr]:
            args.append(r[...])
        outs = refs[k + nf + nr:]
        res = fn(*args)
        if n_out == 1:
            res = (res,)
        for o, v in zip(outs, res):
            o[...] = v

    in_specs = []
    ops = []
    if parts is not None:
        in_specs.append(pl.BlockSpec((1, bn, d), lambda i: (0, i, 0)))
        in_specs.append(pl.BlockSpec((1, bn, d), lambda i: (1, i, 0)))
        ops += [parts, parts]
    for a in fulls:
        in_specs.append(pl.BlockSpec((bn, d), lambda i: (i, 0)))
        ops.append(a)
    for a in rows:
        in_specs.append(pl.BlockSpec((1, d), lambda i: (0, 0)))
        ops.append(a)
    out_shape = [jax.ShapeDtypeStruct((n, d), jnp.float32)] * n_out
    out_specs = [pl.BlockSpec((bn, d), lambda i: (i, 0))] * n_out
    res = pl.pallas_call(
        body, grid=(n // bn,), in_specs=in_specs, out_specs=out_specs,
        out_shape=out_shape)(*ops)
    return res[0] if n_out == 1 else res


def _stats_call(x, degb):
    def body(x_ref, d_ref, o_ref):
        i = pl.program_id(0)
        xb = x_ref[...]
        db = d_ref[...]
        blk = jnp.concatenate([
            jnp.sum(xb, axis=0, keepdims=True),
            jnp.sum(xb * xb, axis=0, keepdims=True),
            jnp.sum(db, axis=0, keepdims=True),
            jnp.sum(db * db, axis=0, keepdims=True),
            jnp.sum((db == 0.0).astype(jnp.float32), axis=0, keepdims=True),
            jnp.zeros((3, 128), jnp.float32),
        ], axis=0)

        @pl.when(i == 0)
        def _():
            o_ref[...] = blk

        @pl.when(i > 0)
        def _():
            o_ref[...] = o_ref[...] + blk

    return pl.pallas_call(
        body,
        grid=(N // BN,),
        in_specs=[pl.BlockSpec((BN, 128), lambda i: (i, 0)),
                  pl.BlockSpec((BN, 128), lambda i: (i, 0))],
        out_specs=pl.BlockSpec((8, 128), lambda i: (0, 0)),
        out_shape=jax.ShapeDtypeStruct((8, 128), jnp.float32),
    )(x, degb)


def _ctrl_call(h0p, W1p, b1, W2p, b2p):
    def body(h_ref, w1_ref, b1_ref, w2_ref, b2_ref, o_ref):
        r1 = lax.dot_general(h_ref[...], w1_ref[...], (((1,), (1,)), ((), ())),
                             preferred_element_type=jnp.float32) + b1_ref[...]
        r1 = jnp.maximum(r1, 0.0)
        o_ref[...] = lax.dot_general(
            r1, w2_ref[...], (((1,), (1,)), ((), ())),
            preferred_element_type=jnp.float32) + b2_ref[...]

    return pl.pallas_call(
        body,
        out_shape=jax.ShapeDtypeStruct((8, 128), jnp.float32),
    )(h0p, W1p, b1, W2p, b2p)


def _edge_score_call(GA, GB, W2, b2):
    def body(c_ref, a_ref, b_ref, w_ref, o_ref):
        r = jnp.maximum(a_ref[...] + b_ref[...], 0.0)
        o_ref[...] = lax.dot_general(
            r, w_ref[...], (((1,), (1,)), ((), ())),
            preferred_element_type=jnp.float32) + c_ref[0]

    return pl.pallas_call(
        body,
        grid=(EP // BE,),
        in_specs=[
            pl.BlockSpec(memory_space=pltpu.SMEM),
            pl.BlockSpec((BE, 128), lambda i: (i, 0)),
            pl.BlockSpec((BE, 128), lambda i: (i, 0)),
            pl.BlockSpec((8, 128), lambda i: (0, 0)),
        ],
        out_specs=pl.BlockSpec((BE, 8), lambda i: (i, 0)),
        out_shape=jax.ShapeDtypeStruct((EP, 8), jnp.float32),
    )(b2, GA, GB, W2)


def _redmax_call(a, mask):
    n, d = a.shape

    def body(a_ref, k_ref, o_ref):
        m = a_ref[...] * k_ref[...] - (1.0 - k_ref[...]) * 1e30
        o_ref[...] = jnp.max(m, axis=0, keepdims=True)

    return pl.pallas_call(
        body,
        out_shape=jax.ShapeDtypeStruct((1, d), jnp.float32))(a, mask)


def _redsumexp_call(a, mxr, mask):
    n, d = a.shape

    def body(a_ref, m_ref, k_ref, o_ref):
        o_ref[...] = jnp.sum(jnp.exp(a_ref[...] - m_ref[...]) * k_ref[...],
                             axis=0, keepdims=True)

    return pl.pallas_call(
        body,
        out_shape=jax.ShapeDtypeStruct((1, d), jnp.float32))(a, mxr, mask)


def _wmul_call(attn1, G):
    def body(a_ref, g_ref, o_ref):
        o_ref[...] = a_ref[...] * g_ref[...]

    return pl.pallas_call(
        body, grid=(EP // BE,),
        in_specs=[pl.BlockSpec((BE, 1), lambda i: (i, 0)),
                  pl.BlockSpec((BE, 128), lambda i: (i, 0))],
        out_specs=pl.BlockSpec((BE, 128), lambda i: (i, 0)),
        out_shape=jax.ShapeDtypeStruct((EP, 128), jnp.float32))(attn1, G)


def _row(v):
    return jnp.full((1, 128), 1.0, jnp.float32) * v


def kernel(edge_index, x, prev_emb, ctrl_W1, ctrl_b1, ctrl_W2, ctrl_b2,
           mode_bias, att_W1, att_b1, att_W2, att_b2, heat_kernels, time_W,
           time_b, scale_weights, sp_W, sp_b, tm_W, tm_b, at_W, at_b, df_W,
           df_b, hr_W, hr_b, out_W1, out_b1, out_W2, out_b2):
    row = edge_index[0]
    col = edge_index[1]
    kpad = jnp.arange(EWP, dtype=jnp.int32)
    padrow = jnp.broadcast_to(N + (kpad % (NP - N - 8)), (NW, EWP))
    padcol = jnp.broadcast_to((kpad * 797) % N, (NW, EWP))
    rowp = padrow.at[:, :EW].set(row.reshape(NW, EW))
    colp = padcol.at[:, :EW].set(col.reshape(NW, EW))
    rowi = rowp.reshape(NW, NCH, CK)
    coli = colp.reshape(NW, NCH, CK)
    rowia = rowp.reshape(NW, NCHA, CKA)
    colia = colp.reshape(NW, NCHA, CKA)
    rowiu = row.reshape(NW, NCHU, CKU)
    coliu = col.reshape(NW, NCHU, CKU)
    ke = jnp.arange(EP, dtype=jnp.int32)
    emask = (ke % EWP < EW).astype(jnp.float32)
    mask2d = emask.reshape(EP // 128, 128)
    zero = jnp.zeros((ZR, D), jnp.float32)

    Sdeg = _spmm_call(jnp.ones((N, D), jnp.float32), rowiu, coliu, zero)
    degb, degm, dinv = _combine(
        Sdeg, [], [],
        lambda p: (p, jnp.maximum(p, 1.0),
                   jnp.maximum(lax.rsqrt(p), 1e-8)),
        n_out=3)

    acc = _stats_call(x, degb)
    s_x = jnp.sum(acc[0])
    s_x2 = jnp.sum(acc[1])
    s_d = acc[2, 0]
    s_d2 = acc[3, 0]
    s_z = acc[4, 0]
    cnt = float(N * D)
    mean_x = s_x / cnt
    std_x = jnp.sqrt(jnp.maximum((s_x2 - cnt * mean_x * mean_x) / (cnt - 1.0),
                                 0.0))
    mean_d = s_d / N
    std_d = jnp.sqrt(jnp.maximum((s_d2 - N * mean_d * mean_d) / (N - 1.0),
                                 0.0))
    stats = jnp.stack([
        jnp.float32(N / 1000.0), jnp.float32(E / max(N, 1)), std_d, s_z / N,
        mean_x, std_x, jnp.float32(1.0), jnp.float32(E / (N * N)),
    ])
    quality = jnp.mean(prev_emb, axis=0)
    h0 = jnp.concatenate([stats, quality])
    h0p = jnp.zeros((8, 256), jnp.float32).at[0, :8 + H].set(h0)
    W1p = jnp.zeros((128, 256), jnp.float32).at[:, :8 + H].set(ctrl_W1)
    W2p = jnp.zeros((128, 128), jnp.float32).at[:5].set(ctrl_W2)
    b2p = jnp.zeros((1, 128), jnp.float32).at[0, :5].set(ctrl_b2)
    logits = _ctrl_call(h0p, W1p, ctrl_b1.reshape(1, 128), W2p, b2p)[0, :5]
    logits = logits + mode_bias
    u = jax.random.uniform(jax.random.key(42), (5,), dtype=jnp.float32)
    g = -jnp.log(-jnp.log(u + 1e-20) + 1e-20)
    probs = jax.nn.softmax((logits + g) / TAU)

    xs = x
    for i in range(L):
        y = _lin(xs, sp_W[i], sp_b[i])
        S = _spmm_call(y, rowiu, coliu, zero)
        xs = _combine(S, [degm], [],
                      lambda p, dm: jnp.maximum(p / dm, 0.0))

    xt = x
    tW = time_W[:, :H]
    for i in range(L):
        xt1 = _lin(xt, tm_W[i], tm_b[i])
        gate = _lin(xt1, tW, time_b, act="sigmoid")
        S = _spmm_call(xt1, rowiu, coliu, zero)
        xt = _combine(S, [xt1, gate, degm], [],
                      lambda p, z, gt, dm: jnp.maximum(
                          gt * z + (1.0 - gt) * (p / dm), 0.0))

    xa = x
    W1a = jnp.zeros((128, H), jnp.float32).at[:64].set(att_W1[:, :H])
    W1b = jnp.zeros((128, H), jnp.float32).at[:64].set(att_W1[:, H:])
    b1p = jnp.zeros((128,), jnp.float32).at[:64].set(att_b1)
    W2p = jnp.zeros((8, 128), jnp.float32).at[:1, :64].set(att_W2)
    for i in range(L):
        xa = _lin(xa, at_W[i], at_b[i])
        A = _lin(xa, W1a, b1p)
        B = _lin(xa, W1b, jnp.zeros((128,), jnp.float32))
        GA, GB, G = _att_gather_call(A, B, xa, rowia, colia)
        sc = _edge_score_call(GA, GB, W2p, att_b2)
        sc2d = sc[:, 0].reshape(EP // 128, 128)
        mx = jnp.max(_redmax_call(sc2d, mask2d))
        ssum = jnp.sum(_redsumexp_call(sc2d, _row(mx), mask2d))
        attn2d = _combine(None, [sc2d, mask2d], [_row(mx), _row(1.0 / ssum)],
                          lambda a, k, m, r: jnp.exp(a - m) * k * r,
                          n=EP // 128, d=128, bn=EP // 128)
        WG = _wmul_call(attn2d.reshape(EP, 1), G)
        S = _scatter_call(WG, rowi, zero)
        xa = _combine(S, [], [], lambda p: jnp.maximum(p, 0.0))

    xd = x
    for i in range(L):
        xd = _lin(xd, df_W[i], df_b[i])
        acc_d = jnp.zeros((N, D), jnp.float32)
        y = _combine(None, [xd, dinv], [], lambda a, b: a * b)
        for k in range(5):
            t = heat_kernels[k]
            S = _spmm_call(y, rowiu, coliu, zero)
            xd, y, acc_d = _combine(
                S, [xd, acc_d, dinv], [_row(t)],
                lambda p, xo, ao, dv, tr: (
                    (1.0 - tr) * xo + tr * (dv * p),
                    dv * ((1.0 - tr) * xo + tr * (dv * p)),
                    ao + (1.0 - tr) * xo + tr * (dv * p)),
                n_out=3)
        xd = _combine(None, [acc_d], [],
                      lambda a: jnp.maximum(a * (1.0 / 5.0), 0.0))

    w3 = jax.nn.softmax(scale_weights)
    xh = x
    for i in range(L):
        xh = _lin(xh, hr_W[i], hr_b[i])
        S1 = _spmm_call(xh, rowiu, coliu, zero)
        h1 = _combine(S1, [degm], [], lambda p, dm: p / dm)
        S2 = _spmm_call(h1, rowiu, coliu, zero)
        h2 = _combine(S2, [degm], [], lambda p, dm: p / dm)
        S3 = _spmm_call(h2, rowiu, coliu, zero)
        xh = _combine(S3, [h1, h2, degm], [_row(w3[0]), _row(w3[1]),
                                           _row(w3[2])],
                      lambda p, a, b, dm, w0, w1, w2: jnp.maximum(
                          w0 * a + w1 * b + w2 * (p / dm), 0.0))

    xout = _combine(None, [xs, xt, xa, xd, xh],
                    [_row(probs[0]), _row(probs[1]), _row(probs[2]),
                     _row(probs[3]), _row(probs[4])],
                    lambda a, b, c, d_, e, p0, p1, p2, p3, p4:
                    p0 * a + p1 * b + p2 * c + p3 * d_ + p4 * e)
    o1 = _lin(xout, out_W1, out_b1, act="relu")
    return _lin(o1, out_W2, out_b2)


# speedup vs baseline: 2.8786x; 1.0694x over previous
import functools

import jax
import jax.numpy as jnp
from jax import lax
from jax.experimental import pallas as pl
from jax.experimental.pallas import tpu as pltpu
from jax.experimental.pallas import tpu_sc as plsc

N = 10000
E = 320000
D = 128
H = 128
OUT = 128
L = 2
TAU = 0.5

NP = 10112
NC = 2
NS = 16
NW = NC * NS
EW = E // NW
CK = 128
EWP = 10240
NCH = EWP // CK
CKA = 128
NCHA = EWP // CKA
CKU = 80
NCHU = EW // CKU
EP = NW * EWP
ZR = NP // NS
PAD_ROW = NP - 8

BN = 1000
BE = 2048


def _sc_mesh():
    return plsc.VectorSubcoreMesh(
        core_axis_name="c", subcore_axis_name="s", num_cores=NC,
        num_subcores=NS)


def _spmm_call(y, rowi, coli, zero):
    d = y.shape[1]

    def body(y_hbm, rowi_hbm, coli_hbm, zero_hbm, out_hbm,
             acc_sh, coli_v, rbuf0, rbuf1, buf0, buf1, sem0, sem1):
        c = lax.axis_index("c")
        s = lax.axis_index("s")
        w = c * NS + s
        pltpu.sync_copy(zero_hbm, acc_sh.at[pl.ds(s * ZR, ZR)])
        pltpu.sync_copy(coli_hbm.at[w], coli_v)
        plsc.subcore_barrier()

        j_first = w * 0
        pltpu.async_copy(y_hbm.at[coli_v.at[j_first]], buf0, sem0)

        def chunk2(g, carry):
            j0 = 2 * g
            j1 = j0 + 1
            j2 = j0 + 2
            pltpu.async_copy(y_hbm.at[coli_v.at[j1]], buf1, sem1)
            pltpu.sync_copy(rowi_hbm.at[w, j0], rbuf0)
            pltpu.make_async_copy(y_hbm.at[pl.ds(0, CK)], buf0, sem0).wait()
            pltpu.sync_copy(buf0, acc_sh.at[rbuf0], add=True)

            @pl.when(j2 < NCH)
            def _():
                pltpu.async_copy(y_hbm.at[coli_v.at[j2]], buf0, sem0)

            pltpu.sync_copy(rowi_hbm.at[w, j1], rbuf1)
            pltpu.make_async_copy(y_hbm.at[pl.ds(0, CK)], buf1, sem1).wait()
            pltpu.sync_copy(buf1, acc_sh.at[rbuf1], add=True)
            return carry

        lax.fori_loop(0, NCH // 2, chunk2, 0)
        plsc.subcore_barrier()
        pltpu.sync_copy(acc_sh.at[pl.ds(s * ZR, ZR)],
                        out_hbm.at[c, pl.ds(s * ZR, ZR)])

    f = pl.kernel(
        body,
        out_type=jax.ShapeDtypeStruct((NC, NP, d), jnp.float32),
        mesh=_sc_mesh(),
        scratch_types=[
            pltpu.VMEM_SHARED((NP, d), jnp.float32),
            pltpu.VMEM((NCH, CK), jnp.int32),
            pltpu.VMEM((CK,), jnp.int32),
            pltpu.VMEM((CK,), jnp.int32),
            pltpu.VMEM((CK, d), jnp.float32),
            pltpu.VMEM((CK, d), jnp.float32),
            pltpu.SemaphoreType.DMA,
            pltpu.SemaphoreType.DMA,
        ],
    )
    return f(y, rowi, coli, zero)


def _att_gather_call(A, B, XA, rowi, coli):
    def body(a_hbm, b_hbm, xa_hbm, rowi_hbm, coli_hbm,
             ga_hbm, gb_hbm, g_hbm,
             rowi_v, coli_v, bufa0, bufb0, bufg0, bufa1, bufb1, bufg1,
             sa0, sb0, sg0, sa1, sb1, sg1):
        c = lax.axis_index("c")
        s = lax.axis_index("s")
        w = c * NS + s
        pltpu.sync_copy(rowi_hbm.at[w], rowi_v)
        pltpu.sync_copy(coli_hbm.at[w], coli_v)

        def issue0(j):
            pltpu.async_copy(a_hbm.at[rowi_v.at[j]], bufa0, sa0)
            pltpu.async_copy(b_hbm.at[coli_v.at[j]], bufb0, sb0)
            pltpu.async_copy(xa_hbm.at[coli_v.at[j]], bufg0, sg0)

        def issue1(j):
            pltpu.async_copy(a_hbm.at[rowi_v.at[j]], bufa1, sa1)
            pltpu.async_copy(b_hbm.at[coli_v.at[j]], bufb1, sb1)
            pltpu.async_copy(xa_hbm.at[coli_v.at[j]], bufg1, sg1)

        j_first = w * 0
        issue0(j_first)

        def chunk2(g, carry):
            j0 = 2 * g
            j1 = j0 + 1
            j2 = j0 + 2
            issue1(j1)
            base0 = pl.multiple_of(w * EWP + j0 * CKA, CKA)
            pltpu.make_async_copy(a_hbm.at[pl.ds(0, CKA)], bufa0, sa0).wait()
            pltpu.sync_copy(bufa0, ga_hbm.at[pl.ds(base0, CKA)])
            pltpu.make_async_copy(b_hbm.at[pl.ds(0, CKA)], bufb0, sb0).wait()
            pltpu.sync_copy(bufb0, gb_hbm.at[pl.ds(base0, CKA)])
            pltpu.make_async_copy(xa_hbm.at[pl.ds(0, CKA)], bufg0, sg0).wait()
            pltpu.sync_copy(bufg0, g_hbm.at[pl.ds(base0, CKA)])

            @pl.when(j2 < NCHA)
            def _():
                issue0(j2)

            base1 = pl.multiple_of(w * EWP + j1 * CKA, CKA)
            pltpu.make_async_copy(a_hbm.at[pl.ds(0, CKA)], bufa1, sa1).wait()
            pltpu.sync_copy(bufa1, ga_hbm.at[pl.ds(base1, CKA)])
            pltpu.make_async_copy(b_hbm.at[pl.ds(0, CKA)], bufb1, sb1).wait()
            pltpu.sync_copy(bufb1, gb_hbm.at[pl.ds(base1, CKA)])
            pltpu.make_async_copy(xa_hbm.at[pl.ds(0, CKA)], bufg1, sg1).wait()
            pltpu.sync_copy(bufg1, g_hbm.at[pl.ds(base1, CKA)])
            return carry

        lax.fori_loop(0, NCHA // 2, chunk2, 0)

    f = pl.kernel(
        body,
        out_type=(jax.ShapeDtypeStruct((EP, D), jnp.float32),
                  jax.ShapeDtypeStruct((EP, D), jnp.float32),
                  jax.ShapeDtypeStruct((EP, D), jnp.float32)),
        mesh=_sc_mesh(),
        scratch_types=[
            pltpu.VMEM((NCHA, CKA), jnp.int32),
            pltpu.VMEM((NCHA, CKA), jnp.int32),
            pltpu.VMEM((CKA, D), jnp.float32),
            pltpu.VMEM((CKA, D), jnp.float32),
            pltpu.VMEM((CKA, D), jnp.float32),
            pltpu.VMEM((CKA, D), jnp.float32),
            pltpu.VMEM((CKA, D), jnp.float32),
            pltpu.VMEM((CKA, D), jnp.float32),
            pltpu.SemaphoreType.DMA,
            pltpu.SemaphoreType.DMA,
            pltpu.SemaphoreType.DMA,
            pltpu.SemaphoreType.DMA,
            pltpu.SemaphoreType.DMA,
            pltpu.SemaphoreType.DMA,
        ],
    )
    return f(A, B, XA, rowi, coli)


def _scatter_call(vals, rowi, zero):
    d = vals.shape[1]

    def body(val_hbm, rowi_hbm, zero_hbm, out_hbm,
             acc_sh, rbuf0, rbuf1, buf0, buf1, sem0, sem1):
        c = lax.axis_index("c")
        s = lax.axis_index("s")
        w = c * NS + s
        pltpu.sync_copy(zero_hbm, acc_sh.at[pl.ds(s * ZR, ZR)])
        plsc.subcore_barrier()

        base_w = pl.multiple_of(w * EWP, CK)
        pltpu.async_copy(val_hbm.at[pl.ds(base_w, CK)], buf0, sem0)

        def chunk2(g, carry):
            j0 = 2 * g
            j1 = j0 + 1
            j2 = j0 + 2
            base1 = pl.multiple_of(w * EWP + j1 * CK, CK)
            pltpu.async_copy(val_hbm.at[pl.ds(base1, CK)], buf1, sem1)
            pltpu.sync_copy(rowi_hbm.at[w, j0], rbuf0)
            pltpu.make_async_copy(val_hbm.at[pl.ds(0, CK)], buf0, sem0).wait()
            pltpu.sync_copy(buf0, acc_sh.at[rbuf0], add=True)

            @pl.when(j2 < NCH)
            def _():
                base2 = pl.multiple_of(w * EWP + j2 * CK, CK)
                pltpu.async_copy(val_hbm.at[pl.ds(base2, CK)], buf0, sem0)

            pltpu.sync_copy(rowi_hbm.at[w, j1], rbuf1)
            pltpu.make_async_copy(val_hbm.at[pl.ds(0, CK)], buf1, sem1).wait()
            pltpu.sync_copy(buf1, acc_sh.at[rbuf1], add=True)
            return carry

        lax.fori_loop(0, NCH // 2, chunk2, 0)
        plsc.subcore_barrier()
        pltpu.sync_copy(acc_sh.at[pl.ds(s * ZR, ZR)],
                        out_hbm.at[c, pl.ds(s * ZR, ZR)])

    f = pl.kernel(
        body,
        out_type=jax.ShapeDtypeStruct((NC, NP, d), jnp.float32),
        mesh=_sc_mesh(),
        scratch_types=[
            pltpu.VMEM_SHARED((NP, d), jnp.float32),
            pltpu.VMEM((CK,), jnp.int32),
            pltpu.VMEM((CK,), jnp.int32),
            pltpu.VMEM((CK, d), jnp.float32),
            pltpu.VMEM((CK, d), jnp.float32),
            pltpu.SemaphoreType.DMA,
            pltpu.SemaphoreType.DMA,
        ],
    )
    return f(vals, rowi, zero)


def _lin(x, W, b, act=None):
    n, din = x.shape
    dout = W.shape[0]
    bn = BN if n == N else BE

    def body(x_ref, w_ref, b_ref, o_ref):
        y = lax.dot_general(x_ref[...], w_ref[...], (((1,), (1,)), ((), ())),
                            preferred_element_type=jnp.float32)
        y = y + b_ref[...]
        if act == "relu":
            y = jnp.maximum(y, 0.0)
        elif act == "sigmoid":
            y = jax.nn.sigmoid(y)
        o_ref[...] = y

    return pl.pallas_call(
        body,
        grid=(n // bn,),
        in_specs=[
            pl.BlockSpec((bn, din), lambda i: (i, 0)),
            pl.BlockSpec((dout, din), lambda i: (0, 0)),
            pl.BlockSpec((1, dout), lambda i: (0, 0)),
        ],
        out_specs=pl.BlockSpec((bn, dout), lambda i: (i, 0)),
        out_shape=jax.ShapeDtypeStruct((n, dout), jnp.float32),
    )(x, W, b.reshape(1, dout))


def _combine(parts, fulls, rows, fn, n=N, d=D, n_out=1, bn=None):
    if bn is None:
        bn = BN if n == N else BE
    nf = len(fulls)
    nr = len(rows)

    def body(*refs):
        k = 0
        args = []
        if parts is not None:
            args.append(refs[0][...][0] + refs[1][...][0])
            k = 2
        for r in refs[k:k + nf + nr]:
            args.append(r[...])
        outs = refs[k + nf + nr:]
        res = fn(*args)
        if n_out == 1:
            res = (res,)
        for o, v in zip(outs, res):
            o[...] = v

    in_specs = []
    ops = []
    if parts is not None:
        in_specs.append(pl.BlockSpec((1, bn, d), lambda i: (0, i, 0)))
        in_specs.append(pl.BlockSpec((1, bn, d), lambda i: (1, i, 0)))
        ops += [parts, parts]
    for a in fulls:
        in_specs.append(pl.BlockSpec((bn, d), lambda i: (i, 0)))
        ops.append(a)
    for a in rows:
        in_specs.append(pl.BlockSpec((1, d), lambda i: (0, 0)))
        ops.append(a)
    out_shape = [jax.ShapeDtypeStruct((n, d), jnp.float32)] * n_out
    out_specs = [pl.BlockSpec((bn, d), lambda i: (i, 0))] * n_out
    res = pl.pallas_call(
        body, grid=(n // bn,), in_specs=in_specs, out_specs=out_specs,
        out_shape=out_shape)(*ops)
    return res[0] if n_out == 1 else res


def _stats_call(x, degb):
    def body(x_ref, d_ref, o_ref):
        i = pl.program_id(0)
        xb = x_ref[...]
        db = d_ref[...]
        blk = jnp.concatenate([
            jnp.sum(xb, axis=0, keepdims=True),
            jnp.sum(xb * xb, axis=0, keepdims=True),
            jnp.sum(db, axis=0, keepdims=True),
            jnp.sum(db * db, axis=0, keepdims=True),
            jnp.sum((db == 0.0).astype(jnp.float32), axis=0, keepdims=True),
            jnp.zeros((3, 128), jnp.float32),
        ], axis=0)

        @pl.when(i == 0)
        def _():
            o_ref[...] = blk

        @pl.when(i > 0)
        def _():
            o_ref[...] = o_ref[...] + blk

    return pl.pallas_call(
        body,
        grid=(N // BN,),
        in_specs=[pl.BlockSpec((BN, 128), lambda i: (i, 0)),
                  pl.BlockSpec((BN, 128), lambda i: (i, 0))],
        out_specs=pl.BlockSpec((8, 128), lambda i: (0, 0)),
        out_shape=jax.ShapeDtypeStruct((8, 128), jnp.float32),
    )(x, degb)


def _ctrl_call(h0p, W1p, b1, W2p, b2p):
    def body(h_ref, w1_ref, b1_ref, w2_ref, b2_ref, o_ref):
        r1 = lax.dot_general(h_ref[...], w1_ref[...], (((1,), (1,)), ((), ())),
                             preferred_element_type=jnp.float32) + b1_ref[...]
        r1 = jnp.maximum(r1, 0.0)
        o_ref[...] = lax.dot_general(
            r1, w2_ref[...], (((1,), (1,)), ((), ())),
            preferred_element_type=jnp.float32) + b2_ref[...]

    return pl.pallas_call(
        body,
        out_shape=jax.ShapeDtypeStruct((8, 128), jnp.float32),
    )(h0p, W1p, b1, W2p, b2p)


def _edge_score_call(GA, GB, W2, b2):
    def body(c_ref, a_ref, b_ref, w_ref, o_ref):
        r = jnp.maximum(a_ref[...] + b_ref[...], 0.0)
        o_ref[...] = lax.dot_general(
            r, w_ref[...], (((1,), (1,)), ((), ())),
            preferred_element_type=jnp.float32) + c_ref[0]

    return pl.pallas_call(
        body,
        grid=(EP // BE,),
        in_specs=[
            pl.BlockSpec(memory_space=pltpu.SMEM),
            pl.BlockSpec((BE, 128), lambda i: (i, 0)),
            pl.BlockSpec((BE, 128), lambda i: (i, 0)),
            pl.BlockSpec((8, 128), lambda i: (0, 0)),
        ],
        out_specs=pl.BlockSpec((BE, 8), lambda i: (i, 0)),
        out_shape=jax.ShapeDtypeStruct((EP, 8), jnp.float32),
    )(b2, GA, GB, W2)


def _redmax_call(a, mask):
    n, d = a.shape

    def body(a_ref, k_ref, o_ref):
        m = a_ref[...] * k_ref[...] - (1.0 - k_ref[...]) * 1e30
        o_ref[...] = jnp.max(m, axis=0, keepdims=True)

    return pl.pallas_call(
        body,
        out_shape=jax.ShapeDtypeStruct((1, d), jnp.float32))(a, mask)


def _redsumexp_call(a, mxr, mask):
    n, d = a.shape

    def body(a_ref, m_ref, k_ref, o_ref):
        o_ref[...] = jnp.sum(jnp.exp(a_ref[...] - m_ref[...]) * k_ref[...],
                             axis=0, keepdims=True)

    return pl.pallas_call(
        body,
        out_shape=jax.ShapeDtypeStruct((1, d), jnp.float32))(a, mxr, mask)


def _wmul_call(attn1, G):
    def body(a_ref, g_ref, o_ref):
        o_ref[...] = a_ref[...] * g_ref[...]

    return pl.pallas_call(
        body, grid=(EP // BE,),
        in_specs=[pl.BlockSpec((BE, 1), lambda i: (i, 0)),
                  pl.BlockSpec((BE, 128), lambda i: (i, 0))],
        out_specs=pl.BlockSpec((BE, 128), lambda i: (i, 0)),
        out_shape=jax.ShapeDtypeStruct((EP, 128), jnp.float32))(attn1, G)


def _row(v):
    return jnp.full((1, 128), 1.0, jnp.float32) * v


def kernel(edge_index, x, prev_emb, ctrl_W1, ctrl_b1, ctrl_W2, ctrl_b2,
           mode_bias, att_W1, att_b1, att_W2, att_b2, heat_kernels, time_W,
           time_b, scale_weights, sp_W, sp_b, tm_W, tm_b, at_W, at_b, df_W,
           df_b, hr_W, hr_b, out_W1, out_b1, out_W2, out_b2):
    row = edge_index[0]
    col = edge_index[1]
    kpad = jnp.arange(EWP, dtype=jnp.int32)
    padrow = jnp.broadcast_to(N + (kpad % (NP - N - 8)), (NW, EWP))
    padcol = jnp.broadcast_to((kpad * 797) % N, (NW, EWP))
    rowp = padrow.at[:, :EW].set(row.reshape(NW, EW))
    colp = padcol.at[:, :EW].set(col.reshape(NW, EW))
    rowi = rowp.reshape(NW, NCH, CK)
    coli = colp.reshape(NW, NCH, CK)
    rowia = rowp.reshape(NW, NCHA, CKA)
    colia = colp.reshape(NW, NCHA, CKA)
    rowiu = row.reshape(NW, NCHU, CKU)
    coliu = col.reshape(NW, NCHU, CKU)
    ke = jnp.arange(EP, dtype=jnp.int32)
    emask = (ke % EWP < EW).astype(jnp.float32)
    mask2d = emask.reshape(EP // 128, 128)
    zero = jnp.zeros((ZR, D), jnp.float32)

    Sdeg = _spmm_call(jnp.ones((N, D), jnp.float32), rowi, coli, zero)
    degb, degm, dinv = _combine(
        Sdeg, [], [],
        lambda p: (p, jnp.maximum(p, 1.0),
                   jnp.maximum(lax.rsqrt(p), 1e-8)),
        n_out=3)

    acc = _stats_call(x, degb)
    s_x = jnp.sum(acc[0])
    s_x2 = jnp.sum(acc[1])
    s_d = acc[2, 0]
    s_d2 = acc[3, 0]
    s_z = acc[4, 0]
    cnt = float(N * D)
    mean_x = s_x / cnt
    std_x = jnp.sqrt(jnp.maximum((s_x2 - cnt * mean_x * mean_x) / (cnt - 1.0),
                                 0.0))
    mean_d = s_d / N
    std_d = jnp.sqrt(jnp.maximum((s_d2 - N * mean_d * mean_d) / (N - 1.0),
                                 0.0))
    stats = jnp.stack([
        jnp.float32(N / 1000.0), jnp.float32(E / max(N, 1)), std_d, s_z / N,
        mean_x, std_x, jnp.float32(1.0), jnp.float32(E / (N * N)),
    ])
    quality = jnp.mean(prev_emb, axis=0)
    h0 = jnp.concatenate([stats, quality])
    h0p = jnp.zeros((8, 256), jnp.float32).at[0, :8 + H].set(h0)
    W1p = jnp.zeros((128, 256), jnp.float32).at[:, :8 + H].set(ctrl_W1)
    W2p = jnp.zeros((128, 128), jnp.float32).at[:5].set(ctrl_W2)
    b2p = jnp.zeros((1, 128), jnp.float32).at[0, :5].set(ctrl_b2)
    logits = _ctrl_call(h0p, W1p, ctrl_b1.reshape(1, 128), W2p, b2p)[0, :5]
    logits = logits + mode_bias
    u = jax.random.uniform(jax.random.key(42), (5,), dtype=jnp.float32)
    g = -jnp.log(-jnp.log(u + 1e-20) + 1e-20)
    probs = jax.nn.softmax((logits + g) / TAU)

    xs = x
    for i in range(L):
        y = _lin(xs, sp_W[i], sp_b[i])
        S = _spmm_call(y, rowi, coli, zero)
        xs = _combine(S, [degm], [],
                      lambda p, dm: jnp.maximum(p / dm, 0.0))

    xt = x
    tW = time_W[:, :H]
    for i in range(L):
        xt1 = _lin(xt, tm_W[i], tm_b[i])
        gate = _lin(xt1, tW, time_b, act="sigmoid")
        S = _spmm_call(xt1, rowi, coli, zero)
        xt = _combine(S, [xt1, gate, degm], [],
                      lambda p, z, gt, dm: jnp.maximum(
                          gt * z + (1.0 - gt) * (p / dm), 0.0))

    xa = x
    W1a = jnp.zeros((128, H), jnp.float32).at[:64].set(att_W1[:, :H])
    W1b = jnp.zeros((128, H), jnp.float32).at[:64].set(att_W1[:, H:])
    b1p = jnp.zeros((128,), jnp.float32).at[:64].set(att_b1)
    W2p = jnp.zeros((8, 128), jnp.float32).at[:1, :64].set(att_W2)
    for i in range(L):
        xa = _lin(xa, at_W[i], at_b[i])
        A = _lin(xa, W1a, b1p)
        B = _lin(xa, W1b, jnp.zeros((128,), jnp.float32))
        GA, GB, G = _att_gather_call(A, B, xa, rowia, colia)
        sc = _edge_score_call(GA, GB, W2p, att_b2)
        sc2d = sc[:, 0].reshape(EP // 128, 128)
        mx = jnp.max(_redmax_call(sc2d, mask2d))
        ssum = jnp.sum(_redsumexp_call(sc2d, _row(mx), mask2d))
        attn2d = _combine(None, [sc2d, mask2d], [_row(mx), _row(1.0 / ssum)],
                          lambda a, k, m, r: jnp.exp(a - m) * k * r,
                          n=EP // 128, d=128, bn=EP // 128)
        WG = _wmul_call(attn2d.reshape(EP, 1), G)
        S = _scatter_call(WG, rowi, zero)
        xa = _combine(S, [], [], lambda p: jnp.maximum(p, 0.0))

    xd = x
    for i in range(L):
        xd = _lin(xd, df_W[i], df_b[i])
        acc_d = jnp.zeros((N, D), jnp.float32)
        y = _combine(None, [xd, dinv], [], lambda a, b: a * b)
        for k in range(5):
            t = heat_kernels[k]
            S = _spmm_call(y, rowi, coli, zero)
            xd, y, acc_d = _combine(
                S, [xd, acc_d, dinv], [_row(t)],
                lambda p, xo, ao, dv, tr: (
                    (1.0 - tr) * xo + tr * (dv * p),
                    dv * ((1.0 - tr) * xo + tr * (dv * p)),
                    ao + (1.0 - tr) * xo + tr * (dv * p)),
                n_out=3)
        xd = _combine(None, [acc_d], [],
                      lambda a: jnp.maximum(a * (1.0 / 5.0), 0.0))

    w3 = jax.nn.softmax(scale_weights)
    xh = x
    for i in range(L):
        xh = _lin(xh, hr_W[i], hr_b[i])
        S1 = _spmm_call(xh, rowi, coli, zero)
        h1 = _combine(S1, [degm], [], lambda p, dm: p / dm)
        S2 = _spmm_call(h1, rowi, coli, zero)
        h2 = _combine(S2, [degm], [], lambda p, dm: p / dm)
        S3 = _spmm_call(h2, rowi, coli, zero)
        xh = _combine(S3, [h1, h2, degm], [_row(w3[0]), _row(w3[1]),
                                           _row(w3[2])],
                      lambda p, a, b, dm, w0, w1, w2: jnp.maximum(
                          w0 * a + w1 * b + w2 * (p / dm), 0.0))

    xout = _combine(None, [xs, xt, xa, xd, xh],
                    [_row(probs[0]), _row(probs[1]), _row(probs[2]),
                     _row(probs[3]), _row(probs[4])],
                    lambda a, b, c, d_, e, p0, p1, p2, p3, p4:
                    p0 * a + p1 * b + p2 * c + p3 * d_ + p4 * e)
    o1 = _lin(xout, out_W1, out_b1, act="relu")
    return _lin(o1, out_W2, out_b2)
